# baseline (speedup 1.0000x reference)
import functools
import math

import jax
import jax.numpy as jnp
from jax import lax
from jax.experimental import pallas as pl
from jax.experimental.pallas import tpu as pltpu
from jax.experimental.pallas import tpu_sc as plsc

N = 10000
E = 320000
Q = 65536
KK = 50
D = 128

NC, NS = 2, 16
NW = NC * NS
EPT = E // NW
QPT = Q // NW
NP = 10240
RPS = NP // NS

CHE = 80
CHQ = 128

BNS = 1.0 / math.sqrt(1.0 + 1e-5)


DB = 4
DGRP = (EPT // CHE) // DB
DREM = EPT // CHE - DGRP * DB
SB = 2
SGRP = (EPT // CHE) // SB
SREM = EPT // CHE - SGRP * SB


def _fill_idx(dstv, dstall, off):
    for j in range(CHE // 16):
        dstv[pl.ds(j * 16, 16)] = dstall[pl.ds(off + j * 16, 16)]


def _deg_body(dst_hbm, z_hbm, ones_hbm, out_hbm, dstv, onesv, degsp, sem):
    c = lax.axis_index("c")
    s = lax.axis_index("s")
    base = (c * NS + s) * EPT
    pltpu.sync_copy(ones_hbm, onesv)
    pltpu.sync_copy(z_hbm.at[pl.ds(s * RPS, RPS)], degsp.at[pl.ds(s * RPS, RPS)])
    plsc.subcore_barrier()

    def step(i, _):
        off = base + i * CHE
        pltpu.sync_copy(dst_hbm.at[pl.ds(off, CHE)], dstv)
        pltpu.sync_copy(onesv, degsp.at[dstv], add=True)
        return ()

    lax.fori_loop(0, EPT // CHE, step, ())
    plsc.subcore_barrier()
    pltpu.sync_copy(degsp.at[pl.ds(s * RPS, RPS)], out_hbm.at[c, pl.ds(s * RPS, RPS)])


@functools.cache
def _sc_mesh():
    return plsc.VectorSubcoreMesh(
        core_axis_name="c", subcore_axis_name="s", num_cores=NC, num_subcores=NS
    )


@functools.cache
def _deg_kernel():
    return pl.kernel(
        _deg_body,
        out_type=jax.ShapeDtypeStruct((NC, NP, D), jnp.float32),
        mesh=_sc_mesh(),
        scratch_types=[
            pltpu.VMEM((CHE,), jnp.int32),
            pltpu.VMEM((CHE, D), jnp.float32),
            pltpu.VMEM_SHARED((NP, D), jnp.float32),
            pltpu.SemaphoreType.DMA,
        ],
    )


def _deg_call(dst, z, ones):
    return _deg_kernel()(dst, z, ones)


def _seg_body(hp_hbm, src_hbm, dst_hbm, z_hbm, out_hbm, srcv, dstv, rows, acc, sem):
    c = lax.axis_index("c")
    s = lax.axis_index("s")
    base = (c * NS + s) * EPT
    pltpu.sync_copy(z_hbm.at[pl.ds(s * RPS, RPS)], acc.at[pl.ds(s * RPS, RPS)])
    plsc.subcore_barrier()

    def step(i, _):
        off = base + i * CHE
        pltpu.sync_copy(src_hbm.at[pl.ds(off, CHE)], srcv)
        pltpu.sync_copy(dst_hbm.at[pl.ds(off, CHE)], dstv)
        pltpu.async_copy(hp_hbm.at[srcv], rows, sem).wait()
        pltpu.sync_copy(rows, acc.at[dstv], add=True)
        return ()

    lax.fori_loop(0, EPT // CHE, step, ())
    plsc.subcore_barrier()
    pltpu.sync_copy(acc.at[pl.ds(s * RPS, RPS)], out_hbm.at[c, pl.ds(s * RPS, RPS)])


@functools.cache
def _seg_kernel():
    return pl.kernel(
        _seg_body,
        out_type=jax.ShapeDtypeStruct((NC, NP, D), jnp.float32),
        mesh=_sc_mesh(),
        scratch_types=[
            pltpu.VMEM((CHE,), jnp.int32),
            pltpu.VMEM((CHE,), jnp.int32),
            pltpu.VMEM((CHE, D), jnp.float32),
            pltpu.VMEM_SHARED((NP, D), jnp.float32),
            pltpu.SemaphoreType.DMA,
        ],
    )


def _seg_call(hp, src, dst, z):
    return _seg_kernel()(hp, src, dst, z)


def _qmul_body(x_hbm, e0_hbm, e1_hbm, out_hbm, i0, i1, r0, r1, sem0, sem1):
    c = lax.axis_index("c")
    s = lax.axis_index("s")
    base = (c * NS + s) * QPT

    def step(i, _):
        off = base + i * CHQ
        pltpu.sync_copy(e0_hbm.at[pl.ds(off, CHQ)], i0)
        pltpu.sync_copy(e1_hbm.at[pl.ds(off, CHQ)], i1)
        cp0 = pltpu.async_copy(x_hbm.at[i0], r0, sem0)
        cp1 = pltpu.async_copy(x_hbm.at[i1], r1, sem1)
        cp0.wait()
        cp1.wait()

        def mulrow(r, _):
            for j in range(D // 16):
                sl = pl.ds(j * 16, 16)
                r0[r, sl] = r0[r, sl] * r1[r, sl]
            return ()

        lax.fori_loop(0, CHQ, mulrow, ())
        pltpu.sync_copy(r0, out_hbm.at[pl.ds(off, CHQ)])
        return ()

    lax.fori_loop(0, QPT // CHQ, step, ())


@functools.cache
def _qmul_kernel():
    return pl.kernel(
        _qmul_body,
        out_type=jax.ShapeDtypeStruct((Q, D), jnp.float32),
        mesh=_sc_mesh(),
        scratch_types=(
            [pltpu.VMEM((CHQ,), jnp.int32)] * 2
            + [pltpu.VMEM((CHQ, D), jnp.float32)] * 2
            + [pltpu.SemaphoreType.DMA] * 2
        ),
    )


def _qmul_call(x, e0, e1):
    return _qmul_kernel()(x, e0, e1)


RB = 2000
GN = N // RB
QB = 2048


def _dis_body(p_ref, out_ref):
    deg = p_ref[0, :, 0] + p_ref[1, :, 0] + 1.0
    out_ref[...] = lax.rsqrt(deg)[:, None]


def _dis(parts):
    return pl.pallas_call(
        _dis_body,
        grid=(GN,),
        in_specs=[pl.BlockSpec((NC, RB, D), lambda i: (0, i, 0))],
        out_specs=pl.BlockSpec((RB, 1), lambda i: (i, 0)),
        out_shape=jax.ShapeDtypeStruct((N, 1), jnp.float32),
    )(parts)


def _pre_body(x_ref, cw_ref, aw_ref, ab_ref, dis_ref,
              hp_ref, tmp_ref, vtz_ref, nf_ref, sums_ref):
    i = pl.program_id(0)
    x = x_ref[...]
    hp_ref[...] = (
        jnp.dot(x, cw_ref[...], preferred_element_type=jnp.float32) * dis_ref[...]
    )
    t = jnp.maximum(
        jnp.dot(x, aw_ref[...], preferred_element_type=jnp.float32)
        + ab_ref[...][None, :],
        0.0,
    )
    tmp_ref[...] = t
    u = t[:, :KK]
    v = t[:, KK:2 * KK]
    z = t[:, 2 * KK:3 * KK]
    vtz = lax.dot_general(
        v, z, (((0,), (0,)), ((), ())), preferred_element_type=jnp.float32
    )
    sums = jnp.stack([jnp.sum(u, axis=0), jnp.sum(v, axis=0)])

    @pl.when(i == 0)
    def _():
        vtz_ref[...] = vtz
        sums_ref[...] = sums

    @pl.when(i > 0)
    def _():
        vtz_ref[...] += vtz
        sums_ref[...] += sums

    @pl.when(i == GN - 1)
    def _():
        stot = sums_ref[...]
        nf_ref[...] = (jnp.sum(stot[0] * stot[1]) / N + 1e-6).reshape(1, 1)


def _pre(x, conv_w, att_w, att_b, dis):
    return pl.pallas_call(
        _pre_body,
        grid=(GN,),
        in_specs=[
            pl.BlockSpec((RB, D), lambda i: (i, 0)),
            pl.BlockSpec((D, D), lambda i: (0, 0)),
            pl.BlockSpec((D, 4 * KK), lambda i: (0, 0)),
            pl.BlockSpec((4 * KK,), lambda i: (0,)),
            pl.BlockSpec((RB, 1), lambda i: (i, 0)),
        ],
        out_specs=[
            pl.BlockSpec((RB, D), lambda i: (i, 0)),
            pl.BlockSpec((RB, 4 * KK), lambda i: (i, 0)),
            pl.BlockSpec((KK, KK), lambda i: (0, 0)),
            pl.BlockSpec((1, 1), lambda i: (0, 0)),
            pl.BlockSpec((2, KK), lambda i: (0, 0)),
        ],
        out_shape=[
            jax.ShapeDtypeStruct((N, D), jnp.float32),
            jax.ShapeDtypeStruct((N, 4 * KK), jnp.float32),
            jax.ShapeDtypeStruct((KK, KK), jnp.float32),
            jax.ShapeDtypeStruct((1, 1), jnp.float32),
            jax.ShapeDtypeStruct((2, KK), jnp.float32),
        ],
    )(x, conv_w, att_w, att_b, dis)


def _mix_body(relu_bn, seg_ref, hp_ref, dis_ref, cb_ref, tmp_ref, vtz_ref,
              nf_ref, dwa_ref, dwb_ref, dwc_ref, db_ref, bng_ref, bnb_ref,
              out_ref):
    seg = seg_ref[0] + seg_ref[1]
    xl = jnp.maximum(
        dis_ref[...] * (seg + hp_ref[...]) + cb_ref[...][None, :], 0.0
    )
    t = tmp_ref[...]
    u = t[:, :KK]
    tt = t[:, 3 * KK:]
    wres = (
        jnp.dot(vtz_ref[...], dwa_ref[...], preferred_element_type=jnp.float32)
        / nf_ref[0, 0]
    )
    y = (
        jnp.dot(u, wres, preferred_element_type=jnp.float32)
        + jnp.dot(tt, dwb_ref[...], preferred_element_type=jnp.float32)
        + jnp.dot(xl, dwc_ref[...], preferred_element_type=jnp.float32)
        + db_ref[...][None, :]
    )
    if relu_bn:
        y = jnp.maximum(y, 0.0) * (bng_ref[...][None, :] * BNS) + bnb_ref[...][None, :]
    out_ref[...] = y


def _mix(segs, hp, dis, conv_b, tmp, vtz, nf, dr_w, dr_b, bn_g, bn_b, relu_bn):
    dwa = dr_w[:KK]
    dwb = dr_w[KK:2 * KK]
    dwc = dr_w[2 * KK:]
    return pl.pallas_call(
        functools.partial(_mix_body, relu_bn),
        grid=(GN,),
        in_specs=[
            pl.BlockSpec((NC, RB, D), lambda i: (0, i, 0)),
            pl.BlockSpec((RB, D), lambda i: (i, 0)),
            pl.BlockSpec((RB, 1), lambda i: (i, 0)),
            pl.BlockSpec((D,), lambda i: (0,)),
            pl.BlockSpec((RB, 4 * KK), lambda i: (i, 0)),
            pl.BlockSpec((KK, KK), lambda i: (0, 0)),
            pl.BlockSpec((1, 1), lambda i: (0, 0)),
            pl.BlockSpec((KK, D), lambda i: (0, 0)),
            pl.BlockSpec((KK, D), lambda i: (0, 0)),
            pl.BlockSpec((D, D), lambda i: (0, 0)),
            pl.BlockSpec((D,), lambda i: (0,)),
            pl.BlockSpec((D,), lambda i: (0,)),
            pl.BlockSpec((D,), lambda i: (0,)),
        ],
        out_specs=pl.BlockSpec((RB, D), lambda i: (i, 0)),
        out_shape=jax.ShapeDtypeStruct((N, D), jnp.float32),
    )(segs, hp, dis, conv_b, tmp, vtz, nf, dwa, dwb, dwc, dr_b, bn_g, bn_b)


def _pred_body(h_ref, w0_ref, b0_ref, w1_ref, b1_ref, out_ref):
    y = jnp.maximum(
        jnp.dot(h_ref[...], w0_ref[...], preferred_element_type=jnp.float32)
        + b0_ref[...][None, :],
        0.0,
    )
    logit = jnp.dot(y, w1_ref[...], preferred_element_type=jnp.float32) + b1_ref[0]
    out_ref[...] = jax.nn.sigmoid(logit)


def _pred(h, w0, b0, w1, b1):
    return pl.pallas_call(
        _pred_body,
        grid=(Q // QB,),
        in_specs=[
            pl.BlockSpec((QB, D), lambda i: (i, 0)),
            pl.BlockSpec((D, D), lambda i: (0, 0)),
            pl.BlockSpec((D,), lambda i: (0,)),
            pl.BlockSpec((D, 1), lambda i: (0, 0)),
            pl.BlockSpec((1,), lambda i: (0,)),
        ],
        out_specs=pl.BlockSpec((QB, 1), lambda i: (i, 0)),
        out_shape=jax.ShapeDtypeStruct((Q, 1), jnp.float32),
    )(h, w0, b0, w1, b1)


def kernel(adj_t, edges, emb, conv_w1, conv_b1, conv_w2, conv_b2, att_w0,
           att_b0, att_w1, att_b1, dr_w0, dr_b0, dr_w1, dr_b1, bn_g, bn_b,
           pred_w0, pred_b0, pred_w1, pred_b1):
    src = adj_t[0]
    dst = adj_t[1]
    e0 = edges[0]
    e1 = edges[1]
    z = jnp.zeros((NP, D), jnp.float32)
    ones = jnp.ones((CHE, D), jnp.float32)

    degp = _deg_call(dst, z, ones)
    dis = _dis(degp)

    hp1, tmp1, vtz1, nf1, _ = _pre(emb, conv_w1, att_w0, att_b0, dis)
    seg1 = _seg_call(hp1, src, dst, z)
    x2 = _mix(seg1, hp1, dis, conv_b1, tmp1, vtz1, nf1,
              dr_w0, dr_b0, bn_g, bn_b, True)

    hp2, tmp2, vtz2, nf2, _ = _pre(x2, conv_w2, att_w1, att_b1, dis)
    seg2 = _seg_call(hp2, src, dst, z)
    x3 = _mix(seg2, hp2, dis, conv_b2, tmp2, vtz2, nf2,
              dr_w1, dr_b1, bn_g, bn_b, False)

    h = _qmul_call(x3, e0, e1)
    return _pred(h, pred_w0, pred_b0, pred_w1, pred_b1)


# speedup vs baseline: 1.6606x; 1.6606x over previous
import functools
import math

import jax
import jax.numpy as jnp
from jax import lax
from jax.experimental import pallas as pl
from jax.experimental.pallas import tpu as pltpu
from jax.experimental.pallas import tpu_sc as plsc

N = 10000
E = 320000
Q = 65536
KK = 50
D = 128

NC, NS = 2, 16
NW = NC * NS
EPT = E // NW
QPT = Q // NW
NP = 10240
RPS = NP // NS

CHE = 80
CHQ = 128

BNS = 1.0 / math.sqrt(1.0 + 1e-5)


DB = 4
DGRP = (EPT // CHE) // DB
DREM = EPT // CHE - DGRP * DB
SB = 2
SGRP = (EPT // CHE) // SB
SREM = EPT // CHE - SGRP * SB


def _fill_idx(dstv, dstall, off):
    for j in range(CHE // 16):
        dstv[pl.ds(j * 16, 16)] = dstall[pl.ds(off + j * 16, 16)]


def _fill_idx(dstv, dstall, off):
    for j in range(CHE // 16):
        dstv[pl.ds(j * 16, 16)] = dstall[pl.ds(off + j * 16, 16)]


def _deg_body(dst_hbm, z_hbm, ones_hbm, out_hbm, dstall, onesv, degsp, lsem, *bufs):
    dstv = bufs[:DB]
    ssem = bufs[DB:]
    c = lax.axis_index("c")
    s = lax.axis_index("s")
    base = (c * NS + s) * EPT
    ldcp = pltpu.async_copy(dst_hbm.at[pl.ds(base, EPT)], dstall, lsem)
    pltpu.sync_copy(ones_hbm, onesv)
    pltpu.sync_copy(z_hbm.at[pl.ds(s * RPS, RPS)], degsp.at[pl.ds(s * RPS, RPS)])
    ldcp.wait()
    plsc.subcore_barrier()

    def group(i, _):
        for k in range(DB):
            _fill_idx(dstv[k], dstall, (i * DB + k) * CHE)
        cps = [
            pltpu.async_copy(onesv, degsp.at[dstv[k]], ssem[k], add=True)
            for k in range(DB)
        ]
        for cp in cps:
            cp.wait()
        return ()

    lax.fori_loop(0, DGRP, group, ())
    for r in range(DREM):
        _fill_idx(dstv[r], dstall, (DGRP * DB + r) * CHE)
        pltpu.sync_copy(onesv, degsp.at[dstv[r]], add=True)
    plsc.subcore_barrier()
    pltpu.sync_copy(degsp.at[pl.ds(s * RPS, RPS)], out_hbm.at[c, pl.ds(s * RPS, RPS)])


@functools.cache
def _sc_mesh():
    return plsc.VectorSubcoreMesh(
        core_axis_name="c", subcore_axis_name="s", num_cores=NC, num_subcores=NS
    )


@functools.cache
def _deg_kernel():
    return pl.kernel(
        _deg_body,
        out_type=jax.ShapeDtypeStruct((NC, NP, D), jnp.float32),
        mesh=_sc_mesh(),
        scratch_types=(
            [
                pltpu.VMEM((EPT,), jnp.int32),
                pltpu.VMEM((CHE, D), jnp.float32),
                pltpu.VMEM_SHARED((NP, D), jnp.float32),
                pltpu.SemaphoreType.DMA,
            ]
            + [pltpu.VMEM((CHE,), jnp.int32)] * DB
            + [pltpu.SemaphoreType.DMA] * DB
        ),
    )


def _deg_call(dst, z, ones):
    return _deg_kernel()(dst, z, ones)


def _seg_body(hp_hbm, src_hbm, dst_hbm, z_hbm, out_hbm,
              srcall, dstall, acc, lsem0, lsem1, *bufs):
    dstv = bufs[:SB]
    rows = bufs[SB:2 * SB]
    gsem = bufs[2 * SB:3 * SB]
    ssem = bufs[3 * SB:]
    c = lax.axis_index("c")
    s = lax.axis_index("s")
    base = (c * NS + s) * EPT
    l0 = pltpu.async_copy(src_hbm.at[pl.ds(base, EPT)], srcall, lsem0)
    l1 = pltpu.async_copy(dst_hbm.at[pl.ds(base, EPT)], dstall, lsem1)
    pltpu.sync_copy(z_hbm.at[pl.ds(s * RPS, RPS)], acc.at[pl.ds(s * RPS, RPS)])
    l0.wait()
    l1.wait()
    plsc.subcore_barrier()

    def group(i, _):
        goff = i * SB * CHE
        gcps = [
            pltpu.async_copy(
                hp_hbm.at[srcall.at[pl.ds(goff + k * CHE, CHE)]], rows[k], gsem[k]
            )
            for k in range(SB)
        ]
        for k in range(SB):
            _fill_idx(dstv[k], dstall, goff + k * CHE)
        scps = []
        for k in range(SB):
            gcps[k].wait()
            scps.append(
                pltpu.async_copy(rows[k], acc.at[dstv[k]], ssem[k], add=True)
            )
        for cp in scps:
            cp.wait()
        return ()

    lax.fori_loop(0, SGRP, group, ())
    for r in range(SREM):
        off = (SGRP * SB + r) * CHE
        pltpu.async_copy(
            hp_hbm.at[srcall.at[pl.ds(off, CHE)]], rows[r], gsem[r]
        ).wait()
        _fill_idx(dstv[r], dstall, off)
        pltpu.sync_copy(rows[r], acc.at[dstv[r]], add=True)
    plsc.subcore_barrier()
    pltpu.sync_copy(acc.at[pl.ds(s * RPS, RPS)], out_hbm.at[c, pl.ds(s * RPS, RPS)])


@functools.cache
def _seg_kernel():
    return pl.kernel(
        _seg_body,
        out_type=jax.ShapeDtypeStruct((NC, NP, D), jnp.float32),
        mesh=_sc_mesh(),
        scratch_types=(
            [
                pltpu.VMEM((EPT,), jnp.int32),
                pltpu.VMEM((EPT,), jnp.int32),
                pltpu.VMEM_SHARED((NP, D), jnp.float32),
                pltpu.SemaphoreType.DMA,
                pltpu.SemaphoreType.DMA,
            ]
            + [pltpu.VMEM((CHE,), jnp.int32)] * SB
            + [pltpu.VMEM((CHE, D), jnp.float32)] * SB
            + [pltpu.SemaphoreType.DMA] * SB
            + [pltpu.SemaphoreType.DMA] * SB
        ),
    )


def _seg_call(hp, src, dst, z):
    return _seg_kernel()(hp, src, dst, z)


def _qmul_body(x_hbm, e0_hbm, e1_hbm, out_hbm, e0all, e1all,
               r0a, r1a, r0b, r1b, lsem0, lsem1, ga0, ga1, gb0, gb1, wsa, wsb):
    c = lax.axis_index("c")
    s = lax.axis_index("s")
    base = (c * NS + s) * QPT
    l0 = pltpu.async_copy(e0_hbm.at[pl.ds(base, QPT)], e0all, lsem0)
    l1 = pltpu.async_copy(e1_hbm.at[pl.ds(base, QPT)], e1all, lsem1)
    l0.wait()
    l1.wait()

    def sg(cidx, r0, r1, s0, s1):
        pltpu.async_copy(x_hbm.at[e0all.at[pl.ds(cidx * CHQ, CHQ)]], r0, s0)
        pltpu.async_copy(x_hbm.at[e1all.at[pl.ds(cidx * CHQ, CHQ)]], r1, s1)

    def wg(cidx, r0, r1, s0, s1):
        pltpu.make_async_copy(
            x_hbm.at[e0all.at[pl.ds(cidx * CHQ, CHQ)]], r0, s0).wait()
        pltpu.make_async_copy(
            x_hbm.at[e1all.at[pl.ds(cidx * CHQ, CHQ)]], r1, s1).wait()

    def mult(r0, r1):
        def mulrow(r, _):
            for j in range(D // 16):
                sl = pl.ds(j * 16, 16)
                r0[r, sl] = r0[r, sl] * r1[r, sl]
            return ()

        lax.fori_loop(0, CHQ, mulrow, ())

    def do_chunk(cidx, r0, r1, s0, s1, ws):
        wg(cidx, r0, r1, s0, s1)
        mult(r0, r1)
        return pltpu.async_copy(r0, out_hbm.at[pl.ds(base + cidx * CHQ, CHQ)], ws)

    nch = QPT // CHQ
    sg(0, r0a, r1a, ga0, ga1)
    sg(1, r0b, r1b, gb0, gb1)

    def it(i, _):
        a = 2 * i
        wa = do_chunk(a, r0a, r1a, ga0, ga1, wsa)
        wb = do_chunk(a + 1, r0b, r1b, gb0, gb1, wsb)
        wa.wait()
        sg(a + 2, r0a, r1a, ga0, ga1)
        wb.wait()
        sg(a + 3, r0b, r1b, gb0, gb1)
        return ()

    lax.fori_loop(0, nch // 2 - 1, it, ())
    a = nch - 2
    wa = do_chunk(a, r0a, r1a, ga0, ga1, wsa)
    wb = do_chunk(a + 1, r0b, r1b, gb0, gb1, wsb)
    wa.wait()
    wb.wait()


@functools.cache
def _qmul_kernel():
    return pl.kernel(
        _qmul_body,
        out_type=jax.ShapeDtypeStruct((Q, D), jnp.float32),
        mesh=_sc_mesh(),
        scratch_types=(
            [pltpu.VMEM((QPT,), jnp.int32)] * 2
            + [pltpu.VMEM((CHQ, D), jnp.float32)] * 4
            + [pltpu.SemaphoreType.DMA] * 8
        ),
    )


def _qmul_call(x, e0, e1):
    return _qmul_kernel()(x, e0, e1)


RB = 2000
GN = N // RB
QB = 2048


def _dis_body(p_ref, out_ref):
    deg = p_ref[0, :, 0] + p_ref[1, :, 0] + 1.0
    out_ref[...] = lax.rsqrt(deg)[:, None]


def _dis(parts):
    return pl.pallas_call(
        _dis_body,
        grid=(GN,),
        in_specs=[pl.BlockSpec((NC, RB, D), lambda i: (0, i, 0))],
        out_specs=pl.BlockSpec((RB, 1), lambda i: (i, 0)),
        out_shape=jax.ShapeDtypeStruct((N, 1), jnp.float32),
    )(parts)


def _pre_body(x_ref, cw_ref, aw_ref, ab_ref, dis_ref,
              hp_ref, tmp_ref, vtz_ref, nf_ref, sums_ref):
    i = pl.program_id(0)
    x = x_ref[...]
    hp_ref[...] = (
        jnp.dot(x, cw_ref[...], preferred_element_type=jnp.float32) * dis_ref[...]
    )
    t = jnp.maximum(
        jnp.dot(x, aw_ref[...], preferred_element_type=jnp.float32)
        + ab_ref[...][None, :],
        0.0,
    )
    tmp_ref[...] = t
    u = t[:, :KK]
    v = t[:, KK:2 * KK]
    z = t[:, 2 * KK:3 * KK]
    vtz = lax.dot_general(
        v, z, (((0,), (0,)), ((), ())), preferred_element_type=jnp.float32
    )
    sums = jnp.stack([jnp.sum(u, axis=0), jnp.sum(v, axis=0)])

    @pl.when(i == 0)
    def _():
        vtz_ref[...] = vtz
        sums_ref[...] = sums

    @pl.when(i > 0)
    def _():
        vtz_ref[...] += vtz
        sums_ref[...] += sums

    @pl.when(i == GN - 1)
    def _():
        stot = sums_ref[...]
        nf_ref[...] = (jnp.sum(stot[0] * stot[1]) / N + 1e-6).reshape(1, 1)


def _pre(x, conv_w, att_w, att_b, dis):
    return pl.pallas_call(
        _pre_body,
        grid=(GN,),
        in_specs=[
            pl.BlockSpec((RB, D), lambda i: (i, 0)),
            pl.BlockSpec((D, D), lambda i: (0, 0)),
            pl.BlockSpec((D, 4 * KK), lambda i: (0, 0)),
            pl.BlockSpec((4 * KK,), lambda i: (0,)),
            pl.BlockSpec((RB, 1), lambda i: (i, 0)),
        ],
        out_specs=[
            pl.BlockSpec((RB, D), lambda i: (i, 0)),
            pl.BlockSpec((RB, 4 * KK), lambda i: (i, 0)),
            pl.BlockSpec((KK, KK), lambda i: (0, 0)),
            pl.BlockSpec((1, 1), lambda i: (0, 0)),
            pl.BlockSpec((2, KK), lambda i: (0, 0)),
        ],
        out_shape=[
            jax.ShapeDtypeStruct((N, D), jnp.float32),
            jax.ShapeDtypeStruct((N, 4 * KK), jnp.float32),
            jax.ShapeDtypeStruct((KK, KK), jnp.float32),
            jax.ShapeDtypeStruct((1, 1), jnp.float32),
            jax.ShapeDtypeStruct((2, KK), jnp.float32),
        ],
    )(x, conv_w, att_w, att_b, dis)


def _mix_body(relu_bn, seg_ref, hp_ref, dis_ref, cb_ref, tmp_ref, vtz_ref,
              nf_ref, dwa_ref, dwb_ref, dwc_ref, db_ref, bng_ref, bnb_ref,
              out_ref):
    seg = seg_ref[0] + seg_ref[1]
    xl = jnp.maximum(
        dis_ref[...] * (seg + hp_ref[...]) + cb_ref[...][None, :], 0.0
    )
    t = tmp_ref[...]
    u = t[:, :KK]
    tt = t[:, 3 * KK:]
    wres = (
        jnp.dot(vtz_ref[...], dwa_ref[...], preferred_element_type=jnp.float32)
        / nf_ref[0, 0]
    )
    y = (
        jnp.dot(u, wres, preferred_element_type=jnp.float32)
        + jnp.dot(tt, dwb_ref[...], preferred_element_type=jnp.float32)
        + jnp.dot(xl, dwc_ref[...], preferred_element_type=jnp.float32)
        + db_ref[...][None, :]
    )
    if relu_bn:
        y = jnp.maximum(y, 0.0) * (bng_ref[...][None, :] * BNS) + bnb_ref[...][None, :]
    out_ref[...] = y


def _mix(segs, hp, dis, conv_b, tmp, vtz, nf, dr_w, dr_b, bn_g, bn_b, relu_bn):
    dwa = dr_w[:KK]
    dwb = dr_w[KK:2 * KK]
    dwc = dr_w[2 * KK:]
    return pl.pallas_call(
        functools.partial(_mix_body, relu_bn),
        grid=(GN,),
        in_specs=[
            pl.BlockSpec((NC, RB, D), lambda i: (0, i, 0)),
            pl.BlockSpec((RB, D), lambda i: (i, 0)),
            pl.BlockSpec((RB, 1), lambda i: (i, 0)),
            pl.BlockSpec((D,), lambda i: (0,)),
            pl.BlockSpec((RB, 4 * KK), lambda i: (i, 0)),
            pl.BlockSpec((KK, KK), lambda i: (0, 0)),
            pl.BlockSpec((1, 1), lambda i: (0, 0)),
            pl.BlockSpec((KK, D), lambda i: (0, 0)),
            pl.BlockSpec((KK, D), lambda i: (0, 0)),
            pl.BlockSpec((D, D), lambda i: (0, 0)),
            pl.BlockSpec((D,), lambda i: (0,)),
            pl.BlockSpec((D,), lambda i: (0,)),
            pl.BlockSpec((D,), lambda i: (0,)),
        ],
        out_specs=pl.BlockSpec((RB, D), lambda i: (i, 0)),
        out_shape=jax.ShapeDtypeStruct((N, D), jnp.float32),
    )(segs, hp, dis, conv_b, tmp, vtz, nf, dwa, dwb, dwc, dr_b, bn_g, bn_b)


def _pred_body(h_ref, w0_ref, b0_ref, w1_ref, b1_ref, out_ref):
    y = jnp.maximum(
        jnp.dot(h_ref[...], w0_ref[...], preferred_element_type=jnp.float32)
        + b0_ref[...][None, :],
        0.0,
    )
    logit = jnp.dot(y, w1_ref[...], preferred_element_type=jnp.float32) + b1_ref[0]
    out_ref[...] = jax.nn.sigmoid(logit)


def _pred(h, w0, b0, w1, b1):
    return pl.pallas_call(
        _pred_body,
        grid=(Q // QB,),
        in_specs=[
            pl.BlockSpec((QB, D), lambda i: (i, 0)),
            pl.BlockSpec((D, D), lambda i: (0, 0)),
            pl.BlockSpec((D,), lambda i: (0,)),
            pl.BlockSpec((D, 1), lambda i: (0, 0)),
            pl.BlockSpec((1,), lambda i: (0,)),
        ],
        out_specs=pl.BlockSpec((QB, 1), lambda i: (i, 0)),
        out_shape=jax.ShapeDtypeStruct((Q, 1), jnp.float32),
    )(h, w0, b0, w1, b1)


def kernel(adj_t, edges, emb, conv_w1, conv_b1, conv_w2, conv_b2, att_w0,
           att_b0, att_w1, att_b1, dr_w0, dr_b0, dr_w1, dr_b1, bn_g, bn_b,
           pred_w0, pred_b0, pred_w1, pred_b1):
    src = adj_t[0]
    dst = adj_t[1]
    e0 = edges[0]
    e1 = edges[1]
    z = jnp.zeros((NP, D), jnp.float32)
    ones = jnp.ones((CHE, D), jnp.float32)

    degp = _deg_call(dst, z, ones)
    dis = _dis(degp)

    hp1, tmp1, vtz1, nf1, _ = _pre(emb, conv_w1, att_w0, att_b0, dis)
    seg1 = _seg_call(hp1, src, dst, z)
    x2 = _mix(seg1, hp1, dis, conv_b1, tmp1, vtz1, nf1,
              dr_w0, dr_b0, bn_g, bn_b, True)

    hp2, tmp2, vtz2, nf2, _ = _pre(x2, conv_w2, att_w1, att_b1, dis)
    seg2 = _seg_call(hp2, src, dst, z)
    x3 = _mix(seg2, hp2, dis, conv_b2, tmp2, vtz2, nf2,
              dr_w1, dr_b1, bn_g, bn_b, False)

    h = _qmul_call(x3, e0, e1)
    return _pred(h, pred_w0, pred_b0, pred_w1, pred_b1)


# speedup vs baseline: 1.6785x; 1.0108x over previous
import functools
import math

import jax
import jax.numpy as jnp
from jax import lax
from jax.experimental import pallas as pl
from jax.experimental.pallas import tpu as pltpu
from jax.experimental.pallas import tpu_sc as plsc

N = 10000
E = 320000
Q = 65536
KK = 50
D = 128

NC, NS = 2, 16
NW = NC * NS
EPT = E // NW
QPT = Q // NW
NP = 10240
RPS = NP // NS

CHE = 80
CHQ = 128

BNS = 1.0 / math.sqrt(1.0 + 1e-5)


DB = 4
DGRP = (EPT // CHE) // DB
DREM = EPT // CHE - DGRP * DB
SB = 2
SGRP = (EPT // CHE) // SB
SREM = EPT // CHE - SGRP * SB


def _fill_idx(dstv, dstall, off):
    for j in range(CHE // 16):
        dstv[pl.ds(j * 16, 16)] = dstall[pl.ds(off + j * 16, 16)]


def _fill_idx(dstv, dstall, off):
    for j in range(CHE // 16):
        dstv[pl.ds(j * 16, 16)] = dstall[pl.ds(off + j * 16, 16)]


def _deg_body(dst_hbm, z_hbm, ones_hbm, out_hbm, dstall, onesv, degsp, lsem, *bufs):
    dstv = bufs[:DB]
    ssem = bufs[DB:]
    c = lax.axis_index("c")
    s = lax.axis_index("s")
    base = (c * NS + s) * EPT
    ldcp = pltpu.async_copy(dst_hbm.at[pl.ds(base, EPT)], dstall, lsem)
    pltpu.sync_copy(ones_hbm, onesv)
    pltpu.sync_copy(z_hbm.at[pl.ds(s * RPS, RPS)], degsp.at[pl.ds(s * RPS, RPS)])
    ldcp.wait()
    plsc.subcore_barrier()

    for k in range(DB):
        _fill_idx(dstv[k], dstall, k * CHE)
        pltpu.async_copy(onesv, degsp.at[dstv[k]], ssem[k], add=True)

    def group(i, _):
        for k in range(DB):
            pltpu.make_async_copy(onesv, degsp.at[dstv[k]], ssem[k]).wait()
            _fill_idx(dstv[k], dstall, ((i + 1) * DB + k) * CHE)
            pltpu.async_copy(onesv, degsp.at[dstv[k]], ssem[k], add=True)
        return ()

    lax.fori_loop(0, DGRP - 1, group, ())
    for k in range(DB):
        pltpu.make_async_copy(onesv, degsp.at[dstv[k]], ssem[k]).wait()
    for r in range(DREM):
        _fill_idx(dstv[r], dstall, (DGRP * DB + r) * CHE)
        pltpu.sync_copy(onesv, degsp.at[dstv[r]], add=True)
    plsc.subcore_barrier()
    pltpu.sync_copy(degsp.at[pl.ds(s * RPS, RPS)], out_hbm.at[c, pl.ds(s * RPS, RPS)])


@functools.cache
def _sc_mesh():
    return plsc.VectorSubcoreMesh(
        core_axis_name="c", subcore_axis_name="s", num_cores=NC, num_subcores=NS
    )


@functools.cache
def _deg_kernel():
    return pl.kernel(
        _deg_body,
        out_type=jax.ShapeDtypeStruct((NC, NP, D), jnp.float32),
        mesh=_sc_mesh(),
        scratch_types=(
            [
                pltpu.VMEM((EPT,), jnp.int32),
                pltpu.VMEM((CHE, D), jnp.float32),
                pltpu.VMEM_SHARED((NP, D), jnp.float32),
                pltpu.SemaphoreType.DMA,
            ]
            + [pltpu.VMEM((CHE,), jnp.int32)] * DB
            + [pltpu.SemaphoreType.DMA] * DB
        ),
    )


def _deg_call(dst, z, ones):
    return _deg_kernel()(dst, z, ones)


def _seg_body(hp_hbm, src_hbm, dst_hbm, z_hbm, out_hbm,
              srcall, dstall, acc, lsem0, lsem1, *bufs):
    dstv = bufs[:SB]
    rows = bufs[SB:2 * SB]
    gsem = bufs[2 * SB:3 * SB]
    ssem = bufs[3 * SB:]
    c = lax.axis_index("c")
    s = lax.axis_index("s")
    base = (c * NS + s) * EPT
    l0 = pltpu.async_copy(src_hbm.at[pl.ds(base, EPT)], srcall, lsem0)
    l1 = pltpu.async_copy(dst_hbm.at[pl.ds(base, EPT)], dstall, lsem1)
    pltpu.sync_copy(z_hbm.at[pl.ds(s * RPS, RPS)], acc.at[pl.ds(s * RPS, RPS)])
    l0.wait()
    l1.wait()
    plsc.subcore_barrier()

    def start_g(cidx, k):
        return pltpu.async_copy(
            hp_hbm.at[srcall.at[pl.ds(cidx * CHE, CHE)]], rows[k], gsem[k])

    def start_s(k):
        return pltpu.async_copy(rows[k], acc.at[dstv[k]], ssem[k], add=True)

    for k in range(SB):
        _fill_idx(dstv[k], dstall, k * CHE)
        start_g(k, k)

    def it(i, _):
        cbase = SB * i
        scps = []
        for k in range(SB):
            pltpu.make_async_copy(
                hp_hbm.at[srcall.at[pl.ds(k * CHE, CHE)]], rows[k], gsem[k]).wait()
            scps.append(start_s(k))
        for k in range(SB):
            scps[k].wait()
            _fill_idx(dstv[k], dstall, (cbase + SB + k) * CHE)
            start_g(cbase + SB + k, k)
        return ()

    lax.fori_loop(0, SGRP - 1, it, ())
    scps = []
    for k in range(SB):
        pltpu.make_async_copy(
            hp_hbm.at[srcall.at[pl.ds(k * CHE, CHE)]], rows[k], gsem[k]).wait()
        scps.append(start_s(k))
    for cp in scps:
        cp.wait()
    for r in range(SREM):
        off = (SGRP * SB + r) * CHE
        _fill_idx(dstv[r], dstall, off)
        start_g(SGRP * SB + r, r).wait()
        pltpu.sync_copy(rows[r], acc.at[dstv[r]], add=True)
    plsc.subcore_barrier()
    pltpu.sync_copy(acc.at[pl.ds(s * RPS, RPS)], out_hbm.at[c, pl.ds(s * RPS, RPS)])


@functools.cache
def _seg_kernel():
    return pl.kernel(
        _seg_body,
        out_type=jax.ShapeDtypeStruct((NC, NP, D), jnp.float32),
        mesh=_sc_mesh(),
        scratch_types=(
            [
                pltpu.VMEM((EPT,), jnp.int32),
                pltpu.VMEM((EPT,), jnp.int32),
                pltpu.VMEM_SHARED((NP, D), jnp.float32),
                pltpu.SemaphoreType.DMA,
                pltpu.SemaphoreType.DMA,
            ]
            + [pltpu.VMEM((CHE,), jnp.int32)] * SB
            + [pltpu.VMEM((CHE, D), jnp.float32)] * SB
            + [pltpu.SemaphoreType.DMA] * SB
            + [pltpu.SemaphoreType.DMA] * SB
        ),
    )


def _seg_call(hp, src, dst, z):
    return _seg_kernel()(hp, src, dst, z)


def _qmul_body(x_hbm, e0_hbm, e1_hbm, out_hbm, e0all, e1all,
               r0a, r1a, r0b, r1b, lsem0, lsem1, ga0, ga1, gb0, gb1, wsa, wsb):
    c = lax.axis_index("c")
    s = lax.axis_index("s")
    base = (c * NS + s) * QPT
    l0 = pltpu.async_copy(e0_hbm.at[pl.ds(base, QPT)], e0all, lsem0)
    l1 = pltpu.async_copy(e1_hbm.at[pl.ds(base, QPT)], e1all, lsem1)
    l0.wait()
    l1.wait()

    def sg(cidx, r0, r1, s0, s1):
        pltpu.async_copy(x_hbm.at[e0all.at[pl.ds(cidx * CHQ, CHQ)]], r0, s0)
        pltpu.async_copy(x_hbm.at[e1all.at[pl.ds(cidx * CHQ, CHQ)]], r1, s1)

    def wg(cidx, r0, r1, s0, s1):
        pltpu.make_async_copy(
            x_hbm.at[e0all.at[pl.ds(cidx * CHQ, CHQ)]], r0, s0).wait()
        pltpu.make_async_copy(
            x_hbm.at[e1all.at[pl.ds(cidx * CHQ, CHQ)]], r1, s1).wait()

    def mult(r0, r1):
        def mulrow(r, _):
            for j in range(D // 16):
                sl = pl.ds(j * 16, 16)
                r0[r, sl] = r0[r, sl] * r1[r, sl]
            return ()

        lax.fori_loop(0, CHQ, mulrow, ())

    def do_chunk(cidx, r0, r1, s0, s1, ws):
        wg(cidx, r0, r1, s0, s1)
        mult(r0, r1)
        return pltpu.async_copy(r0, out_hbm.at[pl.ds(base + cidx * CHQ, CHQ)], ws)

    nch = QPT // CHQ
    sg(0, r0a, r1a, ga0, ga1)
    sg(1, r0b, r1b, gb0, gb1)

    def it(i, _):
        a = 2 * i
        wa = do_chunk(a, r0a, r1a, ga0, ga1, wsa)
        wb = do_chunk(a + 1, r0b, r1b, gb0, gb1, wsb)
        wa.wait()
        sg(a + 2, r0a, r1a, ga0, ga1)
        wb.wait()
        sg(a + 3, r0b, r1b, gb0, gb1)
        return ()

    lax.fori_loop(0, nch // 2 - 1, it, ())
    a = nch - 2
    wa = do_chunk(a, r0a, r1a, ga0, ga1, wsa)
    wb = do_chunk(a + 1, r0b, r1b, gb0, gb1, wsb)
    wa.wait()
    wb.wait()


@functools.cache
def _qmul_kernel():
    return pl.kernel(
        _qmul_body,
        out_type=jax.ShapeDtypeStruct((Q, D), jnp.float32),
        mesh=_sc_mesh(),
        scratch_types=(
            [pltpu.VMEM((QPT,), jnp.int32)] * 2
            + [pltpu.VMEM((CHQ, D), jnp.float32)] * 4
            + [pltpu.SemaphoreType.DMA] * 8
        ),
    )


def _qmul_call(x, e0, e1):
    return _qmul_kernel()(x, e0, e1)


RB = 2000
GN = N // RB
QB = 2048


def _dis_body(p_ref, out_ref):
    deg = p_ref[0, :, 0] + p_ref[1, :, 0] + 1.0
    out_ref[...] = lax.rsqrt(deg)[:, None]


def _dis(parts):
    return pl.pallas_call(
        _dis_body,
        grid=(GN,),
        in_specs=[pl.BlockSpec((NC, RB, D), lambda i: (0, i, 0))],
        out_specs=pl.BlockSpec((RB, 1), lambda i: (i, 0)),
        out_shape=jax.ShapeDtypeStruct((N, 1), jnp.float32),
    )(parts)


def _pre_body(x_ref, cw_ref, aw_ref, ab_ref, dis_ref,
              hp_ref, tmp_ref, vtz_ref, nf_ref, sums_ref):
    i = pl.program_id(0)
    x = x_ref[...]
    hp_ref[...] = (
        jnp.dot(x, cw_ref[...], preferred_element_type=jnp.float32) * dis_ref[...]
    )
    t = jnp.maximum(
        jnp.dot(x, aw_ref[...], preferred_element_type=jnp.float32)
        + ab_ref[...][None, :],
        0.0,
    )
    tmp_ref[...] = t
    u = t[:, :KK]
    v = t[:, KK:2 * KK]
    z = t[:, 2 * KK:3 * KK]
    vtz = lax.dot_general(
        v, z, (((0,), (0,)), ((), ())), preferred_element_type=jnp.float32
    )
    sums = jnp.stack([jnp.sum(u, axis=0), jnp.sum(v, axis=0)])

    @pl.when(i == 0)
    def _():
        vtz_ref[...] = vtz
        sums_ref[...] = sums

    @pl.when(i > 0)
    def _():
        vtz_ref[...] += vtz
        sums_ref[...] += sums

    @pl.when(i == GN - 1)
    def _():
        stot = sums_ref[...]
        nf_ref[...] = (jnp.sum(stot[0] * stot[1]) / N + 1e-6).reshape(1, 1)


def _pre(x, conv_w, att_w, att_b, dis):
    return pl.pallas_call(
        _pre_body,
        grid=(GN,),
        in_specs=[
            pl.BlockSpec((RB, D), lambda i: (i, 0)),
            pl.BlockSpec((D, D), lambda i: (0, 0)),
            pl.BlockSpec((D, 4 * KK), lambda i: (0, 0)),
            pl.BlockSpec((4 * KK,), lambda i: (0,)),
            pl.BlockSpec((RB, 1), lambda i: (i, 0)),
        ],
        out_specs=[
            pl.BlockSpec((RB, D), lambda i: (i, 0)),
            pl.BlockSpec((RB, 4 * KK), lambda i: (i, 0)),
            pl.BlockSpec((KK, KK), lambda i: (0, 0)),
            pl.BlockSpec((1, 1), lambda i: (0, 0)),
            pl.BlockSpec((2, KK), lambda i: (0, 0)),
        ],
        out_shape=[
            jax.ShapeDtypeStruct((N, D), jnp.float32),
            jax.ShapeDtypeStruct((N, 4 * KK), jnp.float32),
            jax.ShapeDtypeStruct((KK, KK), jnp.float32),
            jax.ShapeDtypeStruct((1, 1), jnp.float32),
            jax.ShapeDtypeStruct((2, KK), jnp.float32),
        ],
    )(x, conv_w, att_w, att_b, dis)


def _mix_body(relu_bn, seg_ref, hp_ref, dis_ref, cb_ref, tmp_ref, vtz_ref,
              nf_ref, dwa_ref, dwb_ref, dwc_ref, db_ref, bng_ref, bnb_ref,
              out_ref):
    seg = seg_ref[0] + seg_ref[1]
    xl = jnp.maximum(
        dis_ref[...] * (seg + hp_ref[...]) + cb_ref[...][None, :], 0.0
    )
    t = tmp_ref[...]
    u = t[:, :KK]
    tt = t[:, 3 * KK:]
    wres = (
        jnp.dot(vtz_ref[...], dwa_ref[...], preferred_element_type=jnp.float32)
        / nf_ref[0, 0]
    )
    y = (
        jnp.dot(u, wres, preferred_element_type=jnp.float32)
        + jnp.dot(tt, dwb_ref[...], preferred_element_type=jnp.float32)
        + jnp.dot(xl, dwc_ref[...], preferred_element_type=jnp.float32)
        + db_ref[...][None, :]
    )
    if relu_bn:
        y = jnp.maximum(y, 0.0) * (bng_ref[...][None, :] * BNS) + bnb_ref[...][None, :]
    out_ref[...] = y


def _mix(segs, hp, dis, conv_b, tmp, vtz, nf, dr_w, dr_b, bn_g, bn_b, relu_bn):
    dwa = dr_w[:KK]
    dwb = dr_w[KK:2 * KK]
    dwc = dr_w[2 * KK:]
    return pl.pallas_call(
        functools.partial(_mix_body, relu_bn),
        grid=(GN,),
        in_specs=[
            pl.BlockSpec((NC, RB, D), lambda i: (0, i, 0)),
            pl.BlockSpec((RB, D), lambda i: (i, 0)),
            pl.BlockSpec((RB, 1), lambda i: (i, 0)),
            pl.BlockSpec((D,), lambda i: (0,)),
            pl.BlockSpec((RB, 4 * KK), lambda i: (i, 0)),
            pl.BlockSpec((KK, KK), lambda i: (0, 0)),
            pl.BlockSpec((1, 1), lambda i: (0, 0)),
            pl.BlockSpec((KK, D), lambda i: (0, 0)),
            pl.BlockSpec((KK, D), lambda i: (0, 0)),
            pl.BlockSpec((D, D), lambda i: (0, 0)),
            pl.BlockSpec((D,), lambda i: (0,)),
            pl.BlockSpec((D,), lambda i: (0,)),
            pl.BlockSpec((D,), lambda i: (0,)),
        ],
        out_specs=pl.BlockSpec((RB, D), lambda i: (i, 0)),
        out_shape=jax.ShapeDtypeStruct((N, D), jnp.float32),
    )(segs, hp, dis, conv_b, tmp, vtz, nf, dwa, dwb, dwc, dr_b, bn_g, bn_b)


def _pred_body(h_ref, w0_ref, b0_ref, w1_ref, b1_ref, out_ref):
    y = jnp.maximum(
        jnp.dot(h_ref[...], w0_ref[...], preferred_element_type=jnp.float32)
        + b0_ref[...][None, :],
        0.0,
    )
    logit = jnp.dot(y, w1_ref[...], preferred_element_type=jnp.float32) + b1_ref[0]
    out_ref[...] = jax.nn.sigmoid(logit)


def _pred(h, w0, b0, w1, b1):
    return pl.pallas_call(
        _pred_body,
        grid=(Q // QB,),
        in_specs=[
            pl.BlockSpec((QB, D), lambda i: (i, 0)),
            pl.BlockSpec((D, D), lambda i: (0, 0)),
            pl.BlockSpec((D,), lambda i: (0,)),
            pl.BlockSpec((D, 1), lambda i: (0, 0)),
            pl.BlockSpec((1,), lambda i: (0,)),
        ],
        out_specs=pl.BlockSpec((QB, 1), lambda i: (i, 0)),
        out_shape=jax.ShapeDtypeStruct((Q, 1), jnp.float32),
    )(h, w0, b0, w1, b1)


def kernel(adj_t, edges, emb, conv_w1, conv_b1, conv_w2, conv_b2, att_w0,
           att_b0, att_w1, att_b1, dr_w0, dr_b0, dr_w1, dr_b1, bn_g, bn_b,
           pred_w0, pred_b0, pred_w1, pred_b1):
    src = adj_t[0]
    dst = adj_t[1]
    e0 = edges[0]
    e1 = edges[1]
    z = jnp.zeros((NP, D), jnp.float32)
    ones = jnp.ones((CHE, D), jnp.float32)

    degp = _deg_call(dst, z, ones)
    dis = _dis(degp)

    hp1, tmp1, vtz1, nf1, _ = _pre(emb, conv_w1, att_w0, att_b0, dis)
    seg1 = _seg_call(hp1, src, dst, z)
    x2 = _mix(seg1, hp1, dis, conv_b1, tmp1, vtz1, nf1,
              dr_w0, dr_b0, bn_g, bn_b, True)

    hp2, tmp2, vtz2, nf2, _ = _pre(x2, conv_w2, att_w1, att_b1, dis)
    seg2 = _seg_call(hp2, src, dst, z)
    x3 = _mix(seg2, hp2, dis, conv_b2, tmp2, vtz2, nf2,
              dr_w1, dr_b1, bn_g, bn_b, False)

    h = _qmul_call(x3, e0, e1)
    return _pred(h, pred_w0, pred_b0, pred_w1, pred_b1)


# speedup vs baseline: 1.7768x; 1.0586x over previous
import functools
import math

import jax
import jax.numpy as jnp
from jax import lax
from jax.experimental import pallas as pl
from jax.experimental.pallas import tpu as pltpu
from jax.experimental.pallas import tpu_sc as plsc

N = 10000
E = 320000
Q = 65536
KK = 50
D = 128

NC, NS = 2, 16
NW = NC * NS
EPT = E // NW
QPT = Q // NW
NP = 10240
RPS = NP // NS

CHE = 80
CHQ = 128

BNS = 1.0 / math.sqrt(1.0 + 1e-5)


DB = 4
DGRP = (EPT // CHE) // DB
DREM = EPT // CHE - DGRP * DB
SB = 2
SGRP = (EPT // CHE) // SB
SREM = EPT // CHE - SGRP * SB


def _fill_idx(dstv, dstall, off):
    for j in range(CHE // 16):
        dstv[pl.ds(j * 16, 16)] = dstall[pl.ds(off + j * 16, 16)]


def _fill_idx(dstv, dstall, off):
    for j in range(CHE // 16):
        dstv[pl.ds(j * 16, 16)] = dstall[pl.ds(off + j * 16, 16)]


def _deg_body(dst_hbm, z_hbm, ones_hbm, out_hbm, dstall, onesv, degsp, lsem, *bufs):
    dstv = bufs[:DB]
    ssem = bufs[DB:]
    c = lax.axis_index("c")
    s = lax.axis_index("s")
    base = (c * NS + s) * EPT
    ldcp = pltpu.async_copy(dst_hbm.at[pl.ds(base, EPT)], dstall, lsem)
    pltpu.sync_copy(ones_hbm, onesv)
    pltpu.sync_copy(z_hbm.at[pl.ds(s * RPS, RPS)], degsp.at[pl.ds(s * RPS, RPS)])
    ldcp.wait()
    plsc.subcore_barrier()

    for k in range(DB):
        _fill_idx(dstv[k], dstall, k * CHE)
        pltpu.async_copy(onesv, degsp.at[dstv[k]], ssem[k], add=True)

    def group(i, _):
        for k in range(DB):
            pltpu.make_async_copy(onesv, degsp.at[dstv[k]], ssem[k]).wait()
            _fill_idx(dstv[k], dstall, ((i + 1) * DB + k) * CHE)
            pltpu.async_copy(onesv, degsp.at[dstv[k]], ssem[k], add=True)
        return ()

    lax.fori_loop(0, DGRP - 1, group, ())
    for k in range(DB):
        pltpu.make_async_copy(onesv, degsp.at[dstv[k]], ssem[k]).wait()
    for r in range(DREM):
        _fill_idx(dstv[r], dstall, (DGRP * DB + r) * CHE)
        pltpu.sync_copy(onesv, degsp.at[dstv[r]], add=True)
    plsc.subcore_barrier()
    pltpu.sync_copy(degsp.at[pl.ds(s * RPS, RPS)], out_hbm.at[c, pl.ds(s * RPS, RPS)])


@functools.cache
def _sc_mesh():
    return plsc.VectorSubcoreMesh(
        core_axis_name="c", subcore_axis_name="s", num_cores=NC, num_subcores=NS
    )


@functools.cache
def _deg_kernel():
    return pl.kernel(
        _deg_body,
        out_type=jax.ShapeDtypeStruct((NC, NP, D), jnp.float32),
        mesh=_sc_mesh(),
        scratch_types=(
            [
                pltpu.VMEM((EPT,), jnp.int32),
                pltpu.VMEM((CHE, D), jnp.float32),
                pltpu.VMEM_SHARED((NP, D), jnp.float32),
                pltpu.SemaphoreType.DMA,
            ]
            + [pltpu.VMEM((CHE,), jnp.int32)] * DB
            + [pltpu.SemaphoreType.DMA] * DB
        ),
    )


def _deg_call(dst, z, ones):
    return _deg_kernel()(dst, z, ones)


def _seg_body(hp_hbm, src_hbm, dst_hbm, z_hbm, out_hbm, acc, *bufs):
    srcv = bufs[0:4]
    dstv = bufs[4:8]
    rows = bufs[8:12]
    s1 = bufs[12:16]
    s2 = bufs[16:20]
    gsem = bufs[20:24]
    ssem = bufs[24:28]
    c = lax.axis_index("c")
    s = lax.axis_index("s")
    base = (c * NS + s) * EPT
    pltpu.sync_copy(z_hbm.at[pl.ds(s * RPS, RPS)], acc.at[pl.ds(s * RPS, RPS)])
    plsc.subcore_barrier()

    def idx_issue(cidx, k):
        off = base + cidx * CHE
        pltpu.async_copy(src_hbm.at[pl.ds(off, CHE)], srcv[k], s1[k])
        pltpu.async_copy(dst_hbm.at[pl.ds(off, CHE)], dstv[k], s2[k])

    def idx_wait(k):
        pltpu.make_async_copy(src_hbm.at[pl.ds(base, CHE)], srcv[k], s1[k]).wait()
        pltpu.make_async_copy(dst_hbm.at[pl.ds(base, CHE)], dstv[k], s2[k]).wait()

    def g_issue(k):
        pltpu.async_copy(hp_hbm.at[srcv[k]], rows[k], gsem[k])

    def g_wait(k):
        pltpu.make_async_copy(hp_hbm.at[srcv[k]], rows[k], gsem[k]).wait()

    def s_issue(k):
        pltpu.async_copy(rows[k], acc.at[dstv[k]], ssem[k], add=True)

    def s_wait(k):
        pltpu.make_async_copy(rows[k], acc.at[dstv[k]], ssem[k]).wait()

    for k in range(4):
        idx_issue(k, k)

    def it(i, _):
        for k in range(4):
            idx_wait(k)
            g_issue(k)
        for k in range(4):
            g_wait(k)
            s_issue(k)
        for k in range(4):
            s_wait(k)
            idx_issue((i + 1) * 4 + k, k)
        return ()

    lax.fori_loop(0, DGRP - 1, it, ())
    for k in range(4):
        idx_wait(k)
        g_issue(k)
    for k in range(4):
        g_wait(k)
        s_issue(k)
    for k in range(4):
        s_wait(k)
    for r in range(DREM):
        idx_issue(DGRP * 4 + r, r)
        idx_wait(r)
        g_issue(r)
        g_wait(r)
        pltpu.sync_copy(rows[r], acc.at[dstv[r]], add=True)
    plsc.subcore_barrier()
    pltpu.sync_copy(acc.at[pl.ds(s * RPS, RPS)], out_hbm.at[c, pl.ds(s * RPS, RPS)])


@functools.cache
def _seg_kernel():
    return pl.kernel(
        _seg_body,
        out_type=jax.ShapeDtypeStruct((NC, NP, D), jnp.float32),
        mesh=_sc_mesh(),
        scratch_types=(
            [pltpu.VMEM_SHARED((NP, D), jnp.float32)]
            + [pltpu.VMEM((CHE,), jnp.int32)] * 8
            + [pltpu.VMEM((CHE, D), jnp.float32)] * 4
            + [pltpu.SemaphoreType.DMA] * 16
        ),
    )


def _seg_call(hp, src, dst, z):
    return _seg_kernel()(hp, src, dst, z)


def _qmul_body(x_hbm, e0_hbm, e1_hbm, out_hbm, e0all, e1all,
               r0a, r1a, r0b, r1b, lsem0, lsem1, ga0, ga1, gb0, gb1, wsa, wsb):
    c = lax.axis_index("c")
    s = lax.axis_index("s")
    base = (c * NS + s) * QPT
    l0 = pltpu.async_copy(e0_hbm.at[pl.ds(base, QPT)], e0all, lsem0)
    l1 = pltpu.async_copy(e1_hbm.at[pl.ds(base, QPT)], e1all, lsem1)
    l0.wait()
    l1.wait()

    def sg(cidx, r0, r1, s0, s1):
        pltpu.async_copy(x_hbm.at[e0all.at[pl.ds(cidx * CHQ, CHQ)]], r0, s0)
        pltpu.async_copy(x_hbm.at[e1all.at[pl.ds(cidx * CHQ, CHQ)]], r1, s1)

    def wg(cidx, r0, r1, s0, s1):
        pltpu.make_async_copy(
            x_hbm.at[e0all.at[pl.ds(cidx * CHQ, CHQ)]], r0, s0).wait()
        pltpu.make_async_copy(
            x_hbm.at[e1all.at[pl.ds(cidx * CHQ, CHQ)]], r1, s1).wait()

    def mult(r0, r1):
        def mulrow(r, _):
            for j in range(D // 16):
                sl = pl.ds(j * 16, 16)
                r0[r, sl] = r0[r, sl] * r1[r, sl]
            return ()

        lax.fori_loop(0, CHQ, mulrow, ())

    def do_chunk(cidx, r0, r1, s0, s1, ws):
        wg(cidx, r0, r1, s0, s1)
        mult(r0, r1)
        return pltpu.async_copy(r0, out_hbm.at[pl.ds(base + cidx * CHQ, CHQ)], ws)

    nch = QPT // CHQ
    sg(0, r0a, r1a, ga0, ga1)
    sg(1, r0b, r1b, gb0, gb1)

    def it(i, _):
        a = 2 * i
        wa = do_chunk(a, r0a, r1a, ga0, ga1, wsa)
        wb = do_chunk(a + 1, r0b, r1b, gb0, gb1, wsb)
        wa.wait()
        sg(a + 2, r0a, r1a, ga0, ga1)
        wb.wait()
        sg(a + 3, r0b, r1b, gb0, gb1)
        return ()

    lax.fori_loop(0, nch // 2 - 1, it, ())
    a = nch - 2
    wa = do_chunk(a, r0a, r1a, ga0, ga1, wsa)
    wb = do_chunk(a + 1, r0b, r1b, gb0, gb1, wsb)
    wa.wait()
    wb.wait()


@functools.cache
def _qmul_kernel():
    return pl.kernel(
        _qmul_body,
        out_type=jax.ShapeDtypeStruct((Q, D), jnp.float32),
        mesh=_sc_mesh(),
        scratch_types=(
            [pltpu.VMEM((QPT,), jnp.int32)] * 2
            + [pltpu.VMEM((CHQ, D), jnp.float32)] * 4
            + [pltpu.SemaphoreType.DMA] * 8
        ),
    )


def _qmul_call(x, e0, e1):
    return _qmul_kernel()(x, e0, e1)


RB = 2000
GN = N // RB
QB = 2048


def _dis_body(p_ref, out_ref):
    deg = p_ref[0, :, 0] + p_ref[1, :, 0] + 1.0
    out_ref[...] = lax.rsqrt(deg)[:, None]


def _dis(parts):
    return pl.pallas_call(
        _dis_body,
        grid=(GN,),
        in_specs=[pl.BlockSpec((NC, RB, D), lambda i: (0, i, 0))],
        out_specs=pl.BlockSpec((RB, 1), lambda i: (i, 0)),
        out_shape=jax.ShapeDtypeStruct((N, 1), jnp.float32),
    )(parts)


def _pre_body(x_ref, cw_ref, aw_ref, ab_ref, dis_ref,
              hp_ref, tmp_ref, vtz_ref, nf_ref, sums_ref):
    i = pl.program_id(0)
    x = x_ref[...]
    hp_ref[...] = (
        jnp.dot(x, cw_ref[...], preferred_element_type=jnp.float32) * dis_ref[...]
    )
    t = jnp.maximum(
        jnp.dot(x, aw_ref[...], preferred_element_type=jnp.float32)
        + ab_ref[...][None, :],
        0.0,
    )
    tmp_ref[...] = t
    u = t[:, :KK]
    v = t[:, KK:2 * KK]
    z = t[:, 2 * KK:3 * KK]
    vtz = lax.dot_general(
        v, z, (((0,), (0,)), ((), ())), preferred_element_type=jnp.float32
    )
    sums = jnp.stack([jnp.sum(u, axis=0), jnp.sum(v, axis=0)])

    @pl.when(i == 0)
    def _():
        vtz_ref[...] = vtz
        sums_ref[...] = sums

    @pl.when(i > 0)
    def _():
        vtz_ref[...] += vtz
        sums_ref[...] += sums

    @pl.when(i == GN - 1)
    def _():
        stot = sums_ref[...]
        nf_ref[...] = (jnp.sum(stot[0] * stot[1]) / N + 1e-6).reshape(1, 1)


def _pre(x, conv_w, att_w, att_b, dis):
    return pl.pallas_call(
        _pre_body,
        grid=(GN,),
        in_specs=[
            pl.BlockSpec((RB, D), lambda i: (i, 0)),
            pl.BlockSpec((D, D), lambda i: (0, 0)),
            pl.BlockSpec((D, 4 * KK), lambda i: (0, 0)),
            pl.BlockSpec((4 * KK,), lambda i: (0,)),
            pl.BlockSpec((RB, 1), lambda i: (i, 0)),
        ],
        out_specs=[
            pl.BlockSpec((RB, D), lambda i: (i, 0)),
            pl.BlockSpec((RB, 4 * KK), lambda i: (i, 0)),
            pl.BlockSpec((KK, KK), lambda i: (0, 0)),
            pl.BlockSpec((1, 1), lambda i: (0, 0)),
            pl.BlockSpec((2, KK), lambda i: (0, 0)),
        ],
        out_shape=[
            jax.ShapeDtypeStruct((N, D), jnp.float32),
            jax.ShapeDtypeStruct((N, 4 * KK), jnp.float32),
            jax.ShapeDtypeStruct((KK, KK), jnp.float32),
            jax.ShapeDtypeStruct((1, 1), jnp.float32),
            jax.ShapeDtypeStruct((2, KK), jnp.float32),
        ],
    )(x, conv_w, att_w, att_b, dis)


def _mix_body(relu_bn, seg_ref, hp_ref, dis_ref, cb_ref, tmp_ref, vtz_ref,
              nf_ref, dwa_ref, dwb_ref, dwc_ref, db_ref, bng_ref, bnb_ref,
              out_ref):
    seg = seg_ref[0] + seg_ref[1]
    xl = jnp.maximum(
        dis_ref[...] * (seg + hp_ref[...]) + cb_ref[...][None, :], 0.0
    )
    t = tmp_ref[...]
    u = t[:, :KK]
    tt = t[:, 3 * KK:]
    wres = (
        jnp.dot(vtz_ref[...], dwa_ref[...], preferred_element_type=jnp.float32)
        / nf_ref[0, 0]
    )
    y = (
        jnp.dot(u, wres, preferred_element_type=jnp.float32)
        + jnp.dot(tt, dwb_ref[...], preferred_element_type=jnp.float32)
        + jnp.dot(xl, dwc_ref[...], preferred_element_type=jnp.float32)
        + db_ref[...][None, :]
    )
    if relu_bn:
        y = jnp.maximum(y, 0.0) * (bng_ref[...][None, :] * BNS) + bnb_ref[...][None, :]
    out_ref[...] = y


def _mix(segs, hp, dis, conv_b, tmp, vtz, nf, dr_w, dr_b, bn_g, bn_b, relu_bn):
    dwa = dr_w[:KK]
    dwb = dr_w[KK:2 * KK]
    dwc = dr_w[2 * KK:]
    return pl.pallas_call(
        functools.partial(_mix_body, relu_bn),
        grid=(GN,),
        in_specs=[
            pl.BlockSpec((NC, RB, D), lambda i: (0, i, 0)),
            pl.BlockSpec((RB, D), lambda i: (i, 0)),
            pl.BlockSpec((RB, 1), lambda i: (i, 0)),
            pl.BlockSpec((D,), lambda i: (0,)),
            pl.BlockSpec((RB, 4 * KK), lambda i: (i, 0)),
            pl.BlockSpec((KK, KK), lambda i: (0, 0)),
            pl.BlockSpec((1, 1), lambda i: (0, 0)),
            pl.BlockSpec((KK, D), lambda i: (0, 0)),
            pl.BlockSpec((KK, D), lambda i: (0, 0)),
            pl.BlockSpec((D, D), lambda i: (0, 0)),
            pl.BlockSpec((D,), lambda i: (0,)),
            pl.BlockSpec((D,), lambda i: (0,)),
            pl.BlockSpec((D,), lambda i: (0,)),
        ],
        out_specs=pl.BlockSpec((RB, D), lambda i: (i, 0)),
        out_shape=jax.ShapeDtypeStruct((N, D), jnp.float32),
    )(segs, hp, dis, conv_b, tmp, vtz, nf, dwa, dwb, dwc, dr_b, bn_g, bn_b)


def _pred_body(h_ref, w0_ref, b0_ref, w1_ref, b1_ref, out_ref):
    y = jnp.maximum(
        jnp.dot(h_ref[...], w0_ref[...], preferred_element_type=jnp.float32)
        + b0_ref[...][None, :],
        0.0,
    )
    logit = jnp.dot(y, w1_ref[...], preferred_element_type=jnp.float32) + b1_ref[0]
    out_ref[...] = jax.nn.sigmoid(logit)


def _pred(h, w0, b0, w1, b1):
    return pl.pallas_call(
        _pred_body,
        grid=(Q // QB,),
        in_specs=[
            pl.BlockSpec((QB, D), lambda i: (i, 0)),
            pl.BlockSpec((D, D), lambda i: (0, 0)),
            pl.BlockSpec((D,), lambda i: (0,)),
            pl.BlockSpec((D, 1), lambda i: (0, 0)),
            pl.BlockSpec((1,), lambda i: (0,)),
        ],
        out_specs=pl.BlockSpec((QB, 1), lambda i: (i, 0)),
        out_shape=jax.ShapeDtypeStruct((Q, 1), jnp.float32),
    )(h, w0, b0, w1, b1)


def kernel(adj_t, edges, emb, conv_w1, conv_b1, conv_w2, conv_b2, att_w0,
           att_b0, att_w1, att_b1, dr_w0, dr_b0, dr_w1, dr_b1, bn_g, bn_b,
           pred_w0, pred_b0, pred_w1, pred_b1):
    src = adj_t[0]
    dst = adj_t[1]
    e0 = edges[0]
    e1 = edges[1]
    z = jnp.zeros((NP, D), jnp.float32)
    ones = jnp.ones((CHE, D), jnp.float32)

    degp = _deg_call(dst, z, ones)
    dis = _dis(degp)

    hp1, tmp1, vtz1, nf1, _ = _pre(emb, conv_w1, att_w0, att_b0, dis)
    seg1 = _seg_call(hp1, src, dst, z)
    x2 = _mix(seg1, hp1, dis, conv_b1, tmp1, vtz1, nf1,
              dr_w0, dr_b0, bn_g, bn_b, True)

    hp2, tmp2, vtz2, nf2, _ = _pre(x2, conv_w2, att_w1, att_b1, dis)
    seg2 = _seg_call(hp2, src, dst, z)
    x3 = _mix(seg2, hp2, dis, conv_b2, tmp2, vtz2, nf2,
              dr_w1, dr_b1, bn_g, bn_b, False)

    h = _qmul_call(x3, e0, e1)
    return _pred(h, pred_w0, pred_b0, pred_w1, pred_b1)


# speedup vs baseline: 1.7770x; 1.0001x over previous
import functools
import math

import jax
import jax.numpy as jnp
from jax import lax
from jax.experimental import pallas as pl
from jax.experimental.pallas import tpu as pltpu
from jax.experimental.pallas import tpu_sc as plsc

N = 10000
E = 320000
Q = 65536
KK = 50
D = 128

NC, NS = 2, 16
NW = NC * NS
EPT = E // NW
QPT = Q // NW
NP = 10240
RPS = NP // NS

CHE = 80
CHQ = 128

BNS = 1.0 / math.sqrt(1.0 + 1e-5)


DB = 4
DGRP = (EPT // CHE) // DB
DREM = EPT // CHE - DGRP * DB
SB = 2
SGRP = (EPT // CHE) // SB
SREM = EPT // CHE - SGRP * SB


def _fill_idx(dstv, dstall, off):
    for j in range(CHE // 16):
        dstv[pl.ds(j * 16, 16)] = dstall[pl.ds(off + j * 16, 16)]


def _deg_body(dst_hbm, z_hbm, ones_hbm, out_hbm, dstall, onesv, degsp, lsem, *bufs):
    dstv = bufs[:DB]
    ssem = bufs[DB:]
    c = lax.axis_index("c")
    s = lax.axis_index("s")
    base = (c * NS + s) * EPT
    ldcp = pltpu.async_copy(dst_hbm.at[pl.ds(base, EPT)], dstall, lsem)
    pltpu.sync_copy(ones_hbm, onesv)
    pltpu.sync_copy(z_hbm.at[pl.ds(s * RPS, RPS)], degsp.at[pl.ds(s * RPS, RPS)])
    ldcp.wait()
    plsc.subcore_barrier()

    for k in range(DB):
        _fill_idx(dstv[k], dstall, k * CHE)
        pltpu.async_copy(onesv, degsp.at[dstv[k]], ssem[k], add=True)

    def group(i, _):
        for k in range(DB):
            pltpu.make_async_copy(onesv, degsp.at[dstv[k]], ssem[k]).wait()
            _fill_idx(dstv[k], dstall, ((i + 1) * DB + k) * CHE)
            pltpu.async_copy(onesv, degsp.at[dstv[k]], ssem[k], add=True)
        return ()

    lax.fori_loop(0, DGRP - 1, group, ())
    for k in range(DB):
        pltpu.make_async_copy(onesv, degsp.at[dstv[k]], ssem[k]).wait()
    for r in range(DREM):
        _fill_idx(dstv[r], dstall, (DGRP * DB + r) * CHE)
        pltpu.sync_copy(onesv, degsp.at[dstv[r]], add=True)
    plsc.subcore_barrier()
    pltpu.sync_copy(degsp.at[pl.ds(s * RPS, RPS)], out_hbm.at[c, pl.ds(s * RPS, RPS)])


@functools.cache
def _sc_mesh():
    return plsc.VectorSubcoreMesh(
        core_axis_name="c", subcore_axis_name="s", num_cores=NC, num_subcores=NS
    )


@functools.cache
def _deg_kernel():
    return pl.kernel(
        _deg_body,
        out_type=jax.ShapeDtypeStruct((NC, NP, D), jnp.float32),
        mesh=_sc_mesh(),
        scratch_types=(
            [
                pltpu.VMEM((EPT,), jnp.int32),
                pltpu.VMEM((CHE, D), jnp.float32),
                pltpu.VMEM_SHARED((NP, D), jnp.float32),
                pltpu.SemaphoreType.DMA,
            ]
            + [pltpu.VMEM((CHE,), jnp.int32)] * DB
            + [pltpu.SemaphoreType.DMA] * DB
        ),
    )


def _deg_call(dst, z, ones):
    return _deg_kernel()(dst, z, ones)


def _seg_body(hp_hbm, src_hbm, dst_hbm, z_hbm, out_hbm, acc, *bufs):
    srcv = bufs[0:4]
    dstv = bufs[4:8]
    rows = bufs[8:12]
    s1 = bufs[12:16]
    s2 = bufs[16:20]
    gsem = bufs[20:24]
    ssem = bufs[24:28]
    c = lax.axis_index("c")
    s = lax.axis_index("s")
    base = (c * NS + s) * EPT
    pltpu.sync_copy(z_hbm.at[pl.ds(s * RPS, RPS)], acc.at[pl.ds(s * RPS, RPS)])
    plsc.subcore_barrier()

    def idx_issue(cidx, k):
        off = base + cidx * CHE
        pltpu.async_copy(src_hbm.at[pl.ds(off, CHE)], srcv[k], s1[k])
        pltpu.async_copy(dst_hbm.at[pl.ds(off, CHE)], dstv[k], s2[k])

    def idx_wait(k):
        pltpu.make_async_copy(src_hbm.at[pl.ds(base, CHE)], srcv[k], s1[k]).wait()
        pltpu.make_async_copy(dst_hbm.at[pl.ds(base, CHE)], dstv[k], s2[k]).wait()

    def g_issue(k):
        pltpu.async_copy(hp_hbm.at[srcv[k]], rows[k], gsem[k])

    def g_wait(k):
        pltpu.make_async_copy(hp_hbm.at[srcv[k]], rows[k], gsem[k]).wait()

    def s_issue(k):
        pltpu.async_copy(rows[k], acc.at[dstv[k]], ssem[k], add=True)

    def s_wait(k):
        pltpu.make_async_copy(rows[k], acc.at[dstv[k]], ssem[k]).wait()

    for k in range(4):
        idx_issue(k, k)

    def it(i, _):
        for k in range(4):
            idx_wait(k)
            g_issue(k)
        for k in range(4):
            g_wait(k)
            s_issue(k)
        for k in range(4):
            s_wait(k)
            idx_issue((i + 1) * 4 + k, k)
        return ()

    lax.fori_loop(0, DGRP - 1, it, ())
    for k in range(4):
        idx_wait(k)
        g_issue(k)
    for k in range(4):
        g_wait(k)
        s_issue(k)
    for k in range(4):
        s_wait(k)
    for r in range(DREM):
        idx_issue(DGRP * 4 + r, r)
        idx_wait(r)
        g_issue(r)
        g_wait(r)
        pltpu.sync_copy(rows[r], acc.at[dstv[r]], add=True)
    plsc.subcore_barrier()
    pltpu.sync_copy(acc.at[pl.ds(s * RPS, RPS)], out_hbm.at[c, pl.ds(s * RPS, RPS)])


@functools.cache
def _seg_kernel():
    return pl.kernel(
        _seg_body,
        out_type=jax.ShapeDtypeStruct((NC, NP, D), jnp.float32),
        mesh=_sc_mesh(),
        scratch_types=(
            [pltpu.VMEM_SHARED((NP, D), jnp.float32)]
            + [pltpu.VMEM((CHE,), jnp.int32)] * 8
            + [pltpu.VMEM((CHE, D), jnp.float32)] * 4
            + [pltpu.SemaphoreType.DMA] * 16
        ),
    )


def _seg_call(hp, src, dst, z):
    return _seg_kernel()(hp, src, dst, z)


def _qmul_body(x_hbm, e0_hbm, e1_hbm, out_hbm, e0all, e1all,
               r0a, r1a, r0b, r1b, lsem0, lsem1, ga0, ga1, gb0, gb1, wsa, wsb):
    c = lax.axis_index("c")
    s = lax.axis_index("s")
    base = (c * NS + s) * QPT
    l0 = pltpu.async_copy(e0_hbm.at[pl.ds(base, QPT)], e0all, lsem0)
    l1 = pltpu.async_copy(e1_hbm.at[pl.ds(base, QPT)], e1all, lsem1)
    l0.wait()
    l1.wait()

    def sg(cidx, r0, r1, s0, s1):
        pltpu.async_copy(x_hbm.at[e0all.at[pl.ds(cidx * CHQ, CHQ)]], r0, s0)
        pltpu.async_copy(x_hbm.at[e1all.at[pl.ds(cidx * CHQ, CHQ)]], r1, s1)

    def wg(cidx, r0, r1, s0, s1):
        pltpu.make_async_copy(
            x_hbm.at[e0all.at[pl.ds(cidx * CHQ, CHQ)]], r0, s0).wait()
        pltpu.make_async_copy(
            x_hbm.at[e1all.at[pl.ds(cidx * CHQ, CHQ)]], r1, s1).wait()

    def mult(r0, r1):
        def mulrow(r, _):
            for j in range(D // 16):
                sl = pl.ds(j * 16, 16)
                r0[r, sl] = r0[r, sl] * r1[r, sl]
            return ()

        lax.fori_loop(0, CHQ, mulrow, ())

    def do_chunk(cidx, r0, r1, s0, s1, ws):
        wg(cidx, r0, r1, s0, s1)
        mult(r0, r1)
        return pltpu.async_copy(r0, out_hbm.at[pl.ds(base + cidx * CHQ, CHQ)], ws)

    nch = QPT // CHQ
    sg(0, r0a, r1a, ga0, ga1)
    sg(1, r0b, r1b, gb0, gb1)

    def it(i, _):
        a = 2 * i
        wa = do_chunk(a, r0a, r1a, ga0, ga1, wsa)
        wb = do_chunk(a + 1, r0b, r1b, gb0, gb1, wsb)
        wa.wait()
        sg(a + 2, r0a, r1a, ga0, ga1)
        wb.wait()
        sg(a + 3, r0b, r1b, gb0, gb1)
        return ()

    lax.fori_loop(0, nch // 2 - 1, it, ())
    a = nch - 2
    wa = do_chunk(a, r0a, r1a, ga0, ga1, wsa)
    wb = do_chunk(a + 1, r0b, r1b, gb0, gb1, wsb)
    wa.wait()
    wb.wait()


@functools.cache
def _qmul_kernel():
    return pl.kernel(
        _qmul_body,
        out_type=jax.ShapeDtypeStruct((Q, D), jnp.float32),
        mesh=_sc_mesh(),
        scratch_types=(
            [pltpu.VMEM((QPT,), jnp.int32)] * 2
            + [pltpu.VMEM((CHQ, D), jnp.float32)] * 4
            + [pltpu.SemaphoreType.DMA] * 8
        ),
    )


def _qmul_call(x, e0, e1):
    return _qmul_kernel()(x, e0, e1)


RB = 2000
GN = N // RB
QB = 2048


def _dis_body(p_ref, out_ref):
    deg = p_ref[0, :, 0] + p_ref[1, :, 0] + 1.0
    out_ref[...] = lax.rsqrt(deg)[:, None]


def _dis(parts):
    return pl.pallas_call(
        _dis_body,
        grid=(GN,),
        in_specs=[pl.BlockSpec((NC, RB, D), lambda i: (0, i, 0))],
        out_specs=pl.BlockSpec((RB, 1), lambda i: (i, 0)),
        out_shape=jax.ShapeDtypeStruct((N, 1), jnp.float32),
    )(parts)


def _pre_body(x_ref, cw_ref, aw_ref, ab_ref, dis_ref,
              hp_ref, tmp_ref, vtz_ref, nf_ref, sums_ref):
    i = pl.program_id(0)
    x = x_ref[...]
    hp_ref[...] = (
        jnp.dot(x, cw_ref[...], preferred_element_type=jnp.float32) * dis_ref[...]
    )
    t = jnp.maximum(
        jnp.dot(x, aw_ref[...], preferred_element_type=jnp.float32)
        + ab_ref[...][None, :],
        0.0,
    )
    tmp_ref[...] = t
    u = t[:, :KK]
    v = t[:, KK:2 * KK]
    z = t[:, 2 * KK:3 * KK]
    vtz = lax.dot_general(
        v, z, (((0,), (0,)), ((), ())), preferred_element_type=jnp.float32
    )
    sums = jnp.stack([jnp.sum(u, axis=0), jnp.sum(v, axis=0)])

    @pl.when(i == 0)
    def _():
        vtz_ref[...] = vtz
        sums_ref[...] = sums

    @pl.when(i > 0)
    def _():
        vtz_ref[...] += vtz
        sums_ref[...] += sums

    @pl.when(i == GN - 1)
    def _():
        stot = sums_ref[...]
        nf_ref[...] = (jnp.sum(stot[0] * stot[1]) / N + 1e-6).reshape(1, 1)


def _pre(x, conv_w, att_w, att_b, dis):
    return pl.pallas_call(
        _pre_body,
        grid=(GN,),
        in_specs=[
            pl.BlockSpec((RB, D), lambda i: (i, 0)),
            pl.BlockSpec((D, D), lambda i: (0, 0)),
            pl.BlockSpec((D, 4 * KK), lambda i: (0, 0)),
            pl.BlockSpec((4 * KK,), lambda i: (0,)),
            pl.BlockSpec((RB, 1), lambda i: (i, 0)),
        ],
        out_specs=[
            pl.BlockSpec((RB, D), lambda i: (i, 0)),
            pl.BlockSpec((RB, 4 * KK), lambda i: (i, 0)),
            pl.BlockSpec((KK, KK), lambda i: (0, 0)),
            pl.BlockSpec((1, 1), lambda i: (0, 0)),
            pl.BlockSpec((2, KK), lambda i: (0, 0)),
        ],
        out_shape=[
            jax.ShapeDtypeStruct((N, D), jnp.float32),
            jax.ShapeDtypeStruct((N, 4 * KK), jnp.float32),
            jax.ShapeDtypeStruct((KK, KK), jnp.float32),
            jax.ShapeDtypeStruct((1, 1), jnp.float32),
            jax.ShapeDtypeStruct((2, KK), jnp.float32),
        ],
    )(x, conv_w, att_w, att_b, dis)


def _mix_body(relu_bn, seg_ref, hp_ref, dis_ref, cb_ref, tmp_ref, vtz_ref,
              nf_ref, dwa_ref, dwb_ref, dwc_ref, db_ref, bng_ref, bnb_ref,
              out_ref):
    seg = seg_ref[0] + seg_ref[1]
    xl = jnp.maximum(
        dis_ref[...] * (seg + hp_ref[...]) + cb_ref[...][None, :], 0.0
    )
    t = tmp_ref[...]
    u = t[:, :KK]
    tt = t[:, 3 * KK:]
    wres = (
        jnp.dot(vtz_ref[...], dwa_ref[...], preferred_element_type=jnp.float32)
        / nf_ref[0, 0]
    )
    y = (
        jnp.dot(u, wres, preferred_element_type=jnp.float32)
        + jnp.dot(tt, dwb_ref[...], preferred_element_type=jnp.float32)
        + jnp.dot(xl, dwc_ref[...], preferred_element_type=jnp.float32)
        + db_ref[...][None, :]
    )
    if relu_bn:
        y = jnp.maximum(y, 0.0) * (bng_ref[...][None, :] * BNS) + bnb_ref[...][None, :]
    out_ref[...] = y


def _mix(segs, hp, dis, conv_b, tmp, vtz, nf, dr_w, dr_b, bn_g, bn_b, relu_bn):
    dwa = dr_w[:KK]
    dwb = dr_w[KK:2 * KK]
    dwc = dr_w[2 * KK:]
    return pl.pallas_call(
        functools.partial(_mix_body, relu_bn),
        grid=(GN,),
        in_specs=[
            pl.BlockSpec((NC, RB, D), lambda i: (0, i, 0)),
            pl.BlockSpec((RB, D), lambda i: (i, 0)),
            pl.BlockSpec((RB, 1), lambda i: (i, 0)),
            pl.BlockSpec((D,), lambda i: (0,)),
            pl.BlockSpec((RB, 4 * KK), lambda i: (i, 0)),
            pl.BlockSpec((KK, KK), lambda i: (0, 0)),
            pl.BlockSpec((1, 1), lambda i: (0, 0)),
            pl.BlockSpec((KK, D), lambda i: (0, 0)),
            pl.BlockSpec((KK, D), lambda i: (0, 0)),
            pl.BlockSpec((D, D), lambda i: (0, 0)),
            pl.BlockSpec((D,), lambda i: (0,)),
            pl.BlockSpec((D,), lambda i: (0,)),
            pl.BlockSpec((D,), lambda i: (0,)),
        ],
        out_specs=pl.BlockSpec((RB, D), lambda i: (i, 0)),
        out_shape=jax.ShapeDtypeStruct((N, D), jnp.float32),
    )(segs, hp, dis, conv_b, tmp, vtz, nf, dwa, dwb, dwc, dr_b, bn_g, bn_b)


def _pred_body(h_ref, w0_ref, b0_ref, w1_ref, b1_ref, out_ref):
    y = jnp.maximum(
        jnp.dot(h_ref[...], w0_ref[...], preferred_element_type=jnp.float32)
        + b0_ref[...][None, :],
        0.0,
    )
    logit = jnp.dot(y, w1_ref[...], preferred_element_type=jnp.float32) + b1_ref[0]
    out_ref[...] = jax.nn.sigmoid(logit)


def _pred(h, w0, b0, w1, b1):
    return pl.pallas_call(
        _pred_body,
        grid=(Q // QB,),
        in_specs=[
            pl.BlockSpec((QB, D), lambda i: (i, 0)),
            pl.BlockSpec((D, D), lambda i: (0, 0)),
            pl.BlockSpec((D,), lambda i: (0,)),
            pl.BlockSpec((D, 1), lambda i: (0, 0)),
            pl.BlockSpec((1,), lambda i: (0,)),
        ],
        out_specs=pl.BlockSpec((QB, 1), lambda i: (i, 0)),
        out_shape=jax.ShapeDtypeStruct((Q, 1), jnp.float32),
    )(h, w0, b0, w1, b1)


def kernel(adj_t, edges, emb, conv_w1, conv_b1, conv_w2, conv_b2, att_w0,
           att_b0, att_w1, att_b1, dr_w0, dr_b0, dr_w1, dr_b1, bn_g, bn_b,
           pred_w0, pred_b0, pred_w1, pred_b1):
    src = adj_t[0]
    dst = adj_t[1]
    e0 = edges[0]
    e1 = edges[1]
    z = jnp.zeros((NP, D), jnp.float32)
    ones = jnp.ones((CHE, D), jnp.float32)

    degp = _deg_call(dst, z, ones)
    dis = _dis(degp)

    hp1, tmp1, vtz1, nf1, _ = _pre(emb, conv_w1, att_w0, att_b0, dis)
    seg1 = _seg_call(hp1, src, dst, z)
    x2 = _mix(seg1, hp1, dis, conv_b1, tmp1, vtz1, nf1,
              dr_w0, dr_b0, bn_g, bn_b, True)

    hp2, tmp2, vtz2, nf2, _ = _pre(x2, conv_w2, att_w1, att_b1, dis)
    seg2 = _seg_call(hp2, src, dst, z)
    x3 = _mix(seg2, hp2, dis, conv_b2, tmp2, vtz2, nf2,
              dr_w1, dr_b1, bn_g, bn_b, False)

    h = _qmul_call(x3, e0, e1)
    return _pred(h, pred_w0, pred_b0, pred_w1, pred_b1)


# speedup vs baseline: 1.8372x; 1.0339x over previous
import functools
import math

import jax
import jax.numpy as jnp
from jax import lax
from jax.experimental import pallas as pl
from jax.experimental.pallas import tpu as pltpu
from jax.experimental.pallas import tpu_sc as plsc

N = 10000
E = 320000
Q = 65536
KK = 50
D = 128

NC, NS = 2, 16
NW = NC * NS
EPT = E // NW
QPT = Q // NW
NP = 10240
RPS = NP // NS

CHE = 80
CHQ = 128

BNS = 1.0 / math.sqrt(1.0 + 1e-5)


DB = 4
DGRP = (EPT // CHE) // DB
DREM = EPT // CHE - DGRP * DB
SB = 2
SGRP = (EPT // CHE) // SB
SREM = EPT // CHE - SGRP * SB


def _fill_idx(dstv, dstall, off):
    for j in range(CHE // 16):
        dstv[pl.ds(j * 16, 16)] = dstall[pl.ds(off + j * 16, 16)]


def _deg_body(dst_hbm, z_hbm, ones_hbm, out_hbm, dstall, onesv, degsp, lsem, *bufs):
    dstv = bufs[:DB]
    ssem = bufs[DB:]
    c = lax.axis_index("c")
    s = lax.axis_index("s")
    base = (c * NS + s) * EPT
    ldcp = pltpu.async_copy(dst_hbm.at[pl.ds(base, EPT)], dstall, lsem)
    pltpu.sync_copy(ones_hbm, onesv)
    pltpu.sync_copy(z_hbm.at[pl.ds(s * RPS, RPS)], degsp.at[pl.ds(s * RPS, RPS)])
    ldcp.wait()
    plsc.subcore_barrier()

    for k in range(DB):
        _fill_idx(dstv[k], dstall, k * CHE)
        pltpu.async_copy(onesv, degsp.at[dstv[k]], ssem[k], add=True)

    def group(i, _):
        for k in range(DB):
            pltpu.make_async_copy(onesv, degsp.at[dstv[k]], ssem[k]).wait()
            _fill_idx(dstv[k], dstall, ((i + 1) * DB + k) * CHE)
            pltpu.async_copy(onesv, degsp.at[dstv[k]], ssem[k], add=True)
        return ()

    lax.fori_loop(0, DGRP - 1, group, ())
    for k in range(DB):
        pltpu.make_async_copy(onesv, degsp.at[dstv[k]], ssem[k]).wait()
    for r in range(DREM):
        _fill_idx(dstv[r], dstall, (DGRP * DB + r) * CHE)
        pltpu.sync_copy(onesv, degsp.at[dstv[r]], add=True)
    plsc.subcore_barrier()
    pltpu.sync_copy(degsp.at[pl.ds(s * RPS, RPS)], out_hbm.at[c, pl.ds(s * RPS, RPS)])


@functools.cache
def _sc_mesh():
    return plsc.VectorSubcoreMesh(
        core_axis_name="c", subcore_axis_name="s", num_cores=NC, num_subcores=NS
    )


@functools.cache
def _deg_kernel():
    return pl.kernel(
        _deg_body,
        out_type=jax.ShapeDtypeStruct((NC, NP, D), jnp.float32),
        mesh=_sc_mesh(),
        scratch_types=(
            [
                pltpu.VMEM((EPT,), jnp.int32),
                pltpu.VMEM((CHE, D), jnp.float32),
                pltpu.VMEM_SHARED((NP, D), jnp.float32),
                pltpu.SemaphoreType.DMA,
            ]
            + [pltpu.VMEM((CHE,), jnp.int32)] * DB
            + [pltpu.SemaphoreType.DMA] * DB
        ),
    )


def _deg_call(dst, z, ones):
    return _deg_kernel()(dst, z, ones)


def _seg_body(hp_hbm, src_hbm, dst_hbm, z_hbm, out_hbm, acc, *bufs):
    srcv = bufs[0:4]
    dstv = bufs[4:8]
    rows = bufs[8:12]
    s1 = bufs[12:16]
    s2 = bufs[16:20]
    gsem = bufs[20:24]
    ssem = bufs[24:28]
    c = lax.axis_index("c")
    s = lax.axis_index("s")
    base = (c * NS + s) * EPT
    pltpu.sync_copy(z_hbm.at[pl.ds(s * RPS, RPS)], acc.at[pl.ds(s * RPS, RPS)])
    plsc.subcore_barrier()

    def idx_issue(cidx, k):
        off = base + cidx * CHE
        pltpu.async_copy(src_hbm.at[pl.ds(off, CHE)], srcv[k], s1[k])
        pltpu.async_copy(dst_hbm.at[pl.ds(off, CHE)], dstv[k], s2[k])

    def idx_wait(k):
        pltpu.make_async_copy(src_hbm.at[pl.ds(base, CHE)], srcv[k], s1[k]).wait()
        pltpu.make_async_copy(dst_hbm.at[pl.ds(base, CHE)], dstv[k], s2[k]).wait()

    def g_issue(k):
        pltpu.async_copy(hp_hbm.at[srcv[k]], rows[k], gsem[k])

    def g_wait(k):
        pltpu.make_async_copy(hp_hbm.at[srcv[k]], rows[k], gsem[k]).wait()

    def s_issue(k):
        pltpu.async_copy(rows[k], acc.at[dstv[k]], ssem[k], add=True)

    def s_wait(k):
        pltpu.make_async_copy(rows[k], acc.at[dstv[k]], ssem[k]).wait()

    for k in range(4):
        idx_issue(k, k)

    def it(i, _):
        for k in range(4):
            idx_wait(k)
            g_issue(k)
        for k in range(4):
            g_wait(k)
            s_issue(k)
        for k in range(4):
            s_wait(k)
            idx_issue((i + 1) * 4 + k, k)
        return ()

    lax.fori_loop(0, DGRP - 1, it, ())
    for k in range(4):
        idx_wait(k)
        g_issue(k)
    for k in range(4):
        g_wait(k)
        s_issue(k)
    for k in range(4):
        s_wait(k)
    for r in range(DREM):
        idx_issue(DGRP * 4 + r, r)
        idx_wait(r)
        g_issue(r)
        g_wait(r)
        pltpu.sync_copy(rows[r], acc.at[dstv[r]], add=True)
    plsc.subcore_barrier()
    pltpu.sync_copy(acc.at[pl.ds(s * RPS, RPS)], out_hbm.at[c, pl.ds(s * RPS, RPS)])


@functools.cache
def _seg_kernel():
    return pl.kernel(
        _seg_body,
        out_type=jax.ShapeDtypeStruct((NC, NP, D), jnp.float32),
        mesh=_sc_mesh(),
        scratch_types=(
            [pltpu.VMEM_SHARED((NP, D), jnp.float32)]
            + [pltpu.VMEM((CHE,), jnp.int32)] * 8
            + [pltpu.VMEM((CHE, D), jnp.float32)] * 4
            + [pltpu.SemaphoreType.DMA] * 16
        ),
    )


def _seg_call(hp, src, dst, z):
    return _seg_kernel()(hp, src, dst, z)


def _qmul_body(x_hbm, e0_hbm, e1_hbm, out_hbm, e0all, e1all,
               r0a, r1a, r0b, r1b, lsem0, lsem1, ga0, ga1, gb0, gb1, wsa, wsb):
    c = lax.axis_index("c")
    s = lax.axis_index("s")
    base = (c * NS + s) * QPT
    l0 = pltpu.async_copy(e0_hbm.at[pl.ds(base, QPT)], e0all, lsem0)
    l1 = pltpu.async_copy(e1_hbm.at[pl.ds(base, QPT)], e1all, lsem1)
    l0.wait()
    l1.wait()

    def sg(cidx, r0, r1, s0, s1):
        pltpu.async_copy(x_hbm.at[e0all.at[pl.ds(cidx * CHQ, CHQ)]], r0, s0)
        pltpu.async_copy(x_hbm.at[e1all.at[pl.ds(cidx * CHQ, CHQ)]], r1, s1)

    def wg(cidx, r0, r1, s0, s1):
        pltpu.make_async_copy(
            x_hbm.at[e0all.at[pl.ds(cidx * CHQ, CHQ)]], r0, s0).wait()
        pltpu.make_async_copy(
            x_hbm.at[e1all.at[pl.ds(cidx * CHQ, CHQ)]], r1, s1).wait()

    def mult(r0, r1):
        def mulrow(r, _):
            for j in range(D // 16):
                sl = pl.ds(j * 16, 16)
                r0[r, sl] = r0[r, sl] * r1[r, sl]
            return ()

        lax.fori_loop(0, CHQ, mulrow, ())

    def do_chunk(cidx, r0, r1, s0, s1, ws):
        wg(cidx, r0, r1, s0, s1)
        mult(r0, r1)
        return pltpu.async_copy(r0, out_hbm.at[pl.ds(base + cidx * CHQ, CHQ)], ws)

    nch = QPT // CHQ
    sg(0, r0a, r1a, ga0, ga1)
    sg(1, r0b, r1b, gb0, gb1)

    def it(i, _):
        a = 2 * i
        wa = do_chunk(a, r0a, r1a, ga0, ga1, wsa)
        wb = do_chunk(a + 1, r0b, r1b, gb0, gb1, wsb)
        wa.wait()
        sg(a + 2, r0a, r1a, ga0, ga1)
        wb.wait()
        sg(a + 3, r0b, r1b, gb0, gb1)
        return ()

    lax.fori_loop(0, nch // 2 - 1, it, ())
    a = nch - 2
    wa = do_chunk(a, r0a, r1a, ga0, ga1, wsa)
    wb = do_chunk(a + 1, r0b, r1b, gb0, gb1, wsb)
    wa.wait()
    wb.wait()


@functools.cache
def _qmul_kernel():
    return pl.kernel(
        _qmul_body,
        out_type=jax.ShapeDtypeStruct((Q, D), jnp.float32),
        mesh=_sc_mesh(),
        scratch_types=(
            [pltpu.VMEM((QPT,), jnp.int32)] * 2
            + [pltpu.VMEM((CHQ, D), jnp.float32)] * 4
            + [pltpu.SemaphoreType.DMA] * 8
        ),
    )


def _qmul_call(x, e0, e1):
    return _qmul_kernel()(x, e0, e1)


RB = 2000
GN = N // RB
QB = 2048


def _dis_body(p_ref, out_ref):
    deg = p_ref[0, :, 0] + p_ref[1, :, 0] + 1.0
    out_ref[...] = lax.rsqrt(deg)[:, None]


def _dis(parts):
    return pl.pallas_call(
        _dis_body,
        grid=(GN,),
        in_specs=[pl.BlockSpec((NC, RB, D), lambda i: (0, i, 0))],
        out_specs=pl.BlockSpec((RB, 1), lambda i: (i, 0)),
        out_shape=jax.ShapeDtypeStruct((N, 1), jnp.float32),
    )(parts)


def _pre1_body(x_ref, cw_ref, aw_ref, ab_ref, degp_ref,
               hp_ref, tmp_ref, vtz_ref, nf_ref, sums_ref, dis_ref):
    i = pl.program_id(0)
    dp = degp_ref[...]
    disv = lax.rsqrt(dp[0, :, 0] + dp[1, :, 0] + 1.0)[:, None]
    dis_ref[...] = disv
    x = x_ref[...]
    hp_ref[...] = (
        jnp.dot(x, cw_ref[...], preferred_element_type=jnp.float32) * disv
    )
    t = jnp.maximum(
        jnp.dot(x, aw_ref[...], preferred_element_type=jnp.float32)
        + ab_ref[...][None, :],
        0.0,
    )
    tmp_ref[...] = t
    u = t[:, :KK]
    v = t[:, KK:2 * KK]
    z = t[:, 2 * KK:3 * KK]
    vtz = lax.dot_general(
        v, z, (((0,), (0,)), ((), ())), preferred_element_type=jnp.float32
    )
    sums = jnp.stack([jnp.sum(u, axis=0), jnp.sum(v, axis=0)])

    @pl.when(i == 0)
    def _():
        vtz_ref[...] = vtz
        sums_ref[...] = sums

    @pl.when(i > 0)
    def _():
        vtz_ref[...] += vtz
        sums_ref[...] += sums

    @pl.when(i == GN - 1)
    def _():
        stot = sums_ref[...]
        nf_ref[...] = (jnp.sum(stot[0] * stot[1]) / N + 1e-6).reshape(1, 1)


def _pre1(x, conv_w, att_w, att_b, degp):
    return pl.pallas_call(
        _pre1_body,
        grid=(GN,),
        in_specs=[
            pl.BlockSpec((RB, D), lambda i: (i, 0)),
            pl.BlockSpec((D, D), lambda i: (0, 0)),
            pl.BlockSpec((D, 4 * KK), lambda i: (0, 0)),
            pl.BlockSpec((4 * KK,), lambda i: (0,)),
            pl.BlockSpec((NC, RB, D), lambda i: (0, i, 0)),
        ],
        out_specs=[
            pl.BlockSpec((RB, D), lambda i: (i, 0)),
            pl.BlockSpec((RB, 4 * KK), lambda i: (i, 0)),
            pl.BlockSpec((KK, KK), lambda i: (0, 0)),
            pl.BlockSpec((1, 1), lambda i: (0, 0)),
            pl.BlockSpec((2, KK), lambda i: (0, 0)),
            pl.BlockSpec((RB, 1), lambda i: (i, 0)),
        ],
        out_shape=[
            jax.ShapeDtypeStruct((N, D), jnp.float32),
            jax.ShapeDtypeStruct((N, 4 * KK), jnp.float32),
            jax.ShapeDtypeStruct((KK, KK), jnp.float32),
            jax.ShapeDtypeStruct((1, 1), jnp.float32),
            jax.ShapeDtypeStruct((2, KK), jnp.float32),
            jax.ShapeDtypeStruct((N, 1), jnp.float32),
        ],
    )(x, conv_w, att_w, att_b, degp)


def _pre_body(x_ref, cw_ref, aw_ref, ab_ref, dis_ref,
              hp_ref, tmp_ref, vtz_ref, nf_ref, sums_ref):
    i = pl.program_id(0)
    x = x_ref[...]
    hp_ref[...] = (
        jnp.dot(x, cw_ref[...], preferred_element_type=jnp.float32) * dis_ref[...]
    )
    t = jnp.maximum(
        jnp.dot(x, aw_ref[...], preferred_element_type=jnp.float32)
        + ab_ref[...][None, :],
        0.0,
    )
    tmp_ref[...] = t
    u = t[:, :KK]
    v = t[:, KK:2 * KK]
    z = t[:, 2 * KK:3 * KK]
    vtz = lax.dot_general(
        v, z, (((0,), (0,)), ((), ())), preferred_element_type=jnp.float32
    )
    sums = jnp.stack([jnp.sum(u, axis=0), jnp.sum(v, axis=0)])

    @pl.when(i == 0)
    def _():
        vtz_ref[...] = vtz
        sums_ref[...] = sums

    @pl.when(i > 0)
    def _():
        vtz_ref[...] += vtz
        sums_ref[...] += sums

    @pl.when(i == GN - 1)
    def _():
        stot = sums_ref[...]
        nf_ref[...] = (jnp.sum(stot[0] * stot[1]) / N + 1e-6).reshape(1, 1)


def _pre(x, conv_w, att_w, att_b, dis):
    return pl.pallas_call(
        _pre_body,
        grid=(GN,),
        in_specs=[
            pl.BlockSpec((RB, D), lambda i: (i, 0)),
            pl.BlockSpec((D, D), lambda i: (0, 0)),
            pl.BlockSpec((D, 4 * KK), lambda i: (0, 0)),
            pl.BlockSpec((4 * KK,), lambda i: (0,)),
            pl.BlockSpec((RB, 1), lambda i: (i, 0)),
        ],
        out_specs=[
            pl.BlockSpec((RB, D), lambda i: (i, 0)),
            pl.BlockSpec((RB, 4 * KK), lambda i: (i, 0)),
            pl.BlockSpec((KK, KK), lambda i: (0, 0)),
            pl.BlockSpec((1, 1), lambda i: (0, 0)),
            pl.BlockSpec((2, KK), lambda i: (0, 0)),
        ],
        out_shape=[
            jax.ShapeDtypeStruct((N, D), jnp.float32),
            jax.ShapeDtypeStruct((N, 4 * KK), jnp.float32),
            jax.ShapeDtypeStruct((KK, KK), jnp.float32),
            jax.ShapeDtypeStruct((1, 1), jnp.float32),
            jax.ShapeDtypeStruct((2, KK), jnp.float32),
        ],
    )(x, conv_w, att_w, att_b, dis)


def _mix_body(relu_bn, seg_ref, hp_ref, dis_ref, cb_ref, tmp_ref, vtz_ref,
              nf_ref, dwa_ref, dwb_ref, dwc_ref, db_ref, bng_ref, bnb_ref,
              out_ref):
    seg = seg_ref[0] + seg_ref[1]
    xl = jnp.maximum(
        dis_ref[...] * (seg + hp_ref[...]) + cb_ref[...][None, :], 0.0
    )
    t = tmp_ref[...]
    u = t[:, :KK]
    tt = t[:, 3 * KK:]
    wres = (
        jnp.dot(vtz_ref[...], dwa_ref[...], preferred_element_type=jnp.float32)
        / nf_ref[0, 0]
    )
    y = (
        jnp.dot(u, wres, preferred_element_type=jnp.float32)
        + jnp.dot(tt, dwb_ref[...], preferred_element_type=jnp.float32)
        + jnp.dot(xl, dwc_ref[...], preferred_element_type=jnp.float32)
        + db_ref[...][None, :]
    )
    if relu_bn:
        y = jnp.maximum(y, 0.0) * (bng_ref[...][None, :] * BNS) + bnb_ref[...][None, :]
    out_ref[...] = y


def _mix(segs, hp, dis, conv_b, tmp, vtz, nf, dr_w, dr_b, bn_g, bn_b, relu_bn):
    dwa = dr_w[:KK]
    dwb = dr_w[KK:2 * KK]
    dwc = dr_w[2 * KK:]
    return pl.pallas_call(
        functools.partial(_mix_body, relu_bn),
        grid=(GN,),
        in_specs=[
            pl.BlockSpec((NC, RB, D), lambda i: (0, i, 0)),
            pl.BlockSpec((RB, D), lambda i: (i, 0)),
            pl.BlockSpec((RB, 1), lambda i: (i, 0)),
            pl.BlockSpec((D,), lambda i: (0,)),
            pl.BlockSpec((RB, 4 * KK), lambda i: (i, 0)),
            pl.BlockSpec((KK, KK), lambda i: (0, 0)),
            pl.BlockSpec((1, 1), lambda i: (0, 0)),
            pl.BlockSpec((KK, D), lambda i: (0, 0)),
            pl.BlockSpec((KK, D), lambda i: (0, 0)),
            pl.BlockSpec((D, D), lambda i: (0, 0)),
            pl.BlockSpec((D,), lambda i: (0,)),
            pl.BlockSpec((D,), lambda i: (0,)),
            pl.BlockSpec((D,), lambda i: (0,)),
        ],
        out_specs=pl.BlockSpec((RB, D), lambda i: (i, 0)),
        out_shape=jax.ShapeDtypeStruct((N, D), jnp.float32),
    )(segs, hp, dis, conv_b, tmp, vtz, nf, dwa, dwb, dwc, dr_b, bn_g, bn_b)


def _mixpre_body(seg_ref, hp_ref, dis_ref, cb_ref, tmp_ref, vtz_ref,
                 nf_ref, dwa_ref, dwb_ref, dwc_ref, db_ref, bng_ref, bnb_ref,
                 cw2_ref, aw2_ref, ab2_ref,
                 hp2_ref, tmp2_ref, vtz2_ref, nf2_ref, sums2_ref):
    i = pl.program_id(0)
    seg = seg_ref[0] + seg_ref[1]
    disv = dis_ref[...]
    xl = jnp.maximum(
        disv * (seg + hp_ref[...]) + cb_ref[...][None, :], 0.0
    )
    t = tmp_ref[...]
    u = t[:, :KK]
    tt = t[:, 3 * KK:]
    wres = (
        jnp.dot(vtz_ref[...], dwa_ref[...], preferred_element_type=jnp.float32)
        / nf_ref[0, 0]
    )
    y = (
        jnp.dot(u, wres, preferred_element_type=jnp.float32)
        + jnp.dot(tt, dwb_ref[...], preferred_element_type=jnp.float32)
        + jnp.dot(xl, dwc_ref[...], preferred_element_type=jnp.float32)
        + db_ref[...][None, :]
    )
    y = jnp.maximum(y, 0.0) * (bng_ref[...][None, :] * BNS) + bnb_ref[...][None, :]
    hp2_ref[...] = (
        jnp.dot(y, cw2_ref[...], preferred_element_type=jnp.float32) * disv
    )
    t2 = jnp.maximum(
        jnp.dot(y, aw2_ref[...], preferred_element_type=jnp.float32)
        + ab2_ref[...][None, :],
        0.0,
    )
    tmp2_ref[...] = t2
    u2 = t2[:, :KK]
    v2 = t2[:, KK:2 * KK]
    z2 = t2[:, 2 * KK:3 * KK]
    vtz2 = lax.dot_general(
        v2, z2, (((0,), (0,)), ((), ())), preferred_element_type=jnp.float32
    )
    sums2 = jnp.stack([jnp.sum(u2, axis=0), jnp.sum(v2, axis=0)])

    @pl.when(i == 0)
    def _():
        vtz2_ref[...] = vtz2
        sums2_ref[...] = sums2

    @pl.when(i > 0)
    def _():
        vtz2_ref[...] += vtz2
        sums2_ref[...] += sums2

    @pl.when(i == GN - 1)
    def _():
        stot = sums2_ref[...]
        nf2_ref[...] = (jnp.sum(stot[0] * stot[1]) / N + 1e-6).reshape(1, 1)


def _mixpre(segs, hp, dis, conv_b, tmp, vtz, nf, dr_w, dr_b, bn_g, bn_b,
            cw2, aw2, ab2):
    dwa = dr_w[:KK]
    dwb = dr_w[KK:2 * KK]
    dwc = dr_w[2 * KK:]
    return pl.pallas_call(
        _mixpre_body,
        grid=(GN,),
        in_specs=[
            pl.BlockSpec((NC, RB, D), lambda i: (0, i, 0)),
            pl.BlockSpec((RB, D), lambda i: (i, 0)),
            pl.BlockSpec((RB, 1), lambda i: (i, 0)),
            pl.BlockSpec((D,), lambda i: (0,)),
            pl.BlockSpec((RB, 4 * KK), lambda i: (i, 0)),
            pl.BlockSpec((KK, KK), lambda i: (0, 0)),
            pl.BlockSpec((1, 1), lambda i: (0, 0)),
            pl.BlockSpec((KK, D), lambda i: (0, 0)),
            pl.BlockSpec((KK, D), lambda i: (0, 0)),
            pl.BlockSpec((D, D), lambda i: (0, 0)),
            pl.BlockSpec((D,), lambda i: (0,)),
            pl.BlockSpec((D,), lambda i: (0,)),
            pl.BlockSpec((D,), lambda i: (0,)),
            pl.BlockSpec((D, D), lambda i: (0, 0)),
            pl.BlockSpec((D, 4 * KK), lambda i: (0, 0)),
            pl.BlockSpec((4 * KK,), lambda i: (0,)),
        ],
        out_specs=[
            pl.BlockSpec((RB, D), lambda i: (i, 0)),
            pl.BlockSpec((RB, 4 * KK), lambda i: (i, 0)),
            pl.BlockSpec((KK, KK), lambda i: (0, 0)),
            pl.BlockSpec((1, 1), lambda i: (0, 0)),
            pl.BlockSpec((2, KK), lambda i: (0, 0)),
        ],
        out_shape=[
            jax.ShapeDtypeStruct((N, D), jnp.float32),
            jax.ShapeDtypeStruct((N, 4 * KK), jnp.float32),
            jax.ShapeDtypeStruct((KK, KK), jnp.float32),
            jax.ShapeDtypeStruct((1, 1), jnp.float32),
            jax.ShapeDtypeStruct((2, KK), jnp.float32),
        ],
    )(segs, hp, dis, conv_b, tmp, vtz, nf, dwa, dwb, dwc, dr_b, bn_g, bn_b,
      cw2, aw2, ab2)


def _pred_body(h_ref, w0_ref, b0_ref, w1_ref, b1_ref, out_ref):
    y = jnp.maximum(
        jnp.dot(h_ref[...], w0_ref[...], preferred_element_type=jnp.float32)
        + b0_ref[...][None, :],
        0.0,
    )
    logit = jnp.dot(y, w1_ref[...], preferred_element_type=jnp.float32) + b1_ref[0]
    out_ref[...] = jax.nn.sigmoid(logit)


def _pred(h, w0, b0, w1, b1):
    return pl.pallas_call(
        _pred_body,
        grid=(Q // QB,),
        in_specs=[
            pl.BlockSpec((QB, D), lambda i: (i, 0)),
            pl.BlockSpec((D, D), lambda i: (0, 0)),
            pl.BlockSpec((D,), lambda i: (0,)),
            pl.BlockSpec((D, 1), lambda i: (0, 0)),
            pl.BlockSpec((1,), lambda i: (0,)),
        ],
        out_specs=pl.BlockSpec((QB, 1), lambda i: (i, 0)),
        out_shape=jax.ShapeDtypeStruct((Q, 1), jnp.float32),
    )(h, w0, b0, w1, b1)


def kernel(adj_t, edges, emb, conv_w1, conv_b1, conv_w2, conv_b2, att_w0,
           att_b0, att_w1, att_b1, dr_w0, dr_b0, dr_w1, dr_b1, bn_g, bn_b,
           pred_w0, pred_b0, pred_w1, pred_b1):
    src = adj_t[0]
    dst = adj_t[1]
    e0 = edges[0]
    e1 = edges[1]
    z = jnp.zeros((NP, D), jnp.float32)
    ones = jnp.ones((CHE, D), jnp.float32)

    degp = _deg_call(dst, z, ones)

    hp1, tmp1, vtz1, nf1, _, dis = _pre1(emb, conv_w1, att_w0, att_b0, degp)
    seg1 = _seg_call(hp1, src, dst, z)
    hp2, tmp2, vtz2, nf2, _ = _mixpre(seg1, hp1, dis, conv_b1, tmp1, vtz1,
                                      nf1, dr_w0, dr_b0, bn_g, bn_b,
                                      conv_w2, att_w1, att_b1)
    seg2 = _seg_call(hp2, src, dst, z)
    x3 = _mix(seg2, hp2, dis, conv_b2, tmp2, vtz2, nf2,
              dr_w1, dr_b1, bn_g, bn_b, False)

    h = _qmul_call(x3, e0, e1)
    return _pred(h, pred_w0, pred_b0, pred_w1, pred_b1)
